# Initial kernel scaffold; baseline (speedup 1.0000x reference)
import jax
import jax.numpy as jnp
from jax.experimental import pallas as pl


def kernel(x_src, x_dst, edge_distance, edge_src, edge_dst, Wr1, br1, Wr2, br2, W1, W0, b0, ln_g, ln_b, alpha_dot, W2, Wp, bp):
    raise NotImplementedError("write your pallas kernel here")


# speedup vs baseline: 6.9468x; 6.9468x over previous
import functools

import jax
import jax.numpy as jnp
import numpy as np
from jax.experimental import pallas as pl

N = 10000
E = 160000
L2 = 9
C = 32
H = 32
HEADS = 8
A = 16
V = 8
OUT = 32
EDGE = 64
EXTRA = HEADS * A + H
L_OF_PY = (0, 1, 1, 1, 2, 2, 2, 2, 2)

EB = 1280
NB = 1024
ROW = 640
N_PAD = 10240


def _edge_kernel(ed_ref, gs_ref, gd_ref, wr1_ref, br1_ref, wr2_ref, br2_ref,
                 w1_ref, w0_ref, b0_ref, lng_ref, lnb_ref, proj_ref, ad_ref,
                 exp8_ref, w2_ref, out_ref):
    f32 = jnp.float32
    dot = functools.partial(jnp.dot, preferred_element_type=f32)

    ed = ed_ref[...]
    rad = dot(ed, wr1_ref[...]) + br1_ref[...]
    rad = rad * jax.nn.sigmoid(rad)
    gate = dot(rad, wr2_ref[...]) + br2_ref[...]

    gs0 = gs_ref[:, 0:C]
    gd0 = gd_ref[:, 0:C]
    extra = dot(gs0, w0_ref[0:C, :]) + dot(gd0, w0_ref[C:2 * C, :]) + b0_ref[...]
    gating = extra[:, HEADS * A:]
    sg = jax.nn.sigmoid(gating)
    scal = gating * sg

    xa = extra[:, :HEADS * A]
    proj = proj_ref[...]
    mu = dot(xa, proj)
    cent = xa - mu
    var = dot(cent * cent, proj)
    xan = cent * jax.lax.rsqrt(var + 1e-5) * lng_ref[...] + lnb_ref[...]
    xsl = xan * jax.nn.sigmoid(xan) * 0.8 + 0.2 * xan
    alpha = dot(xsl, ad_ref[...])
    s = jnp.exp(alpha)
    s64 = dot(s, exp8_ref[...])

    for i in range(L2):
        l = L_OF_PY[i]
        if i == 0:
            hid = scal
        else:
            gsi = gs_ref[:, i * C:(i + 1) * C]
            gdi = gd_ref[:, i * C:(i + 1) * C]
            hid = (dot(gsi, w1_ref[l, 0:C, :]) + dot(gdi, w1_ref[l, C:2 * C, :]))
            hid = hid * gate * sg
        val = dot(hid, w2_ref[l])
        out_ref[:, i * 64:(i + 1) * 64] = val * s64
    out_ref[:, 576:584] = s
    out_ref[:, 584:ROW] = jnp.zeros((ed.shape[0], ROW - 584), f32)


def _node_kernel(node_ref, wp_ref, bp_ref, exp8_ref, out_ref):
    f32 = jnp.float32
    dot = functools.partial(jnp.dot, preferred_element_type=f32)
    asum = node_ref[:, 576:584]
    inv = 1.0 / (asum + 1e-16)
    inv64 = dot(inv, exp8_ref[...])
    for i in range(L2):
        l = L_OF_PY[i]
        acc = node_ref[:, i * 64:(i + 1) * 64] * inv64
        o = dot(acc, wp_ref[l])
        if i == 0:
            o = o + bp_ref[...]
        out_ref[:, i * OUT:(i + 1) * OUT] = o


def _run_edge_kernel(ed, gs, gd, Wr1, br1, Wr2, br2, W1, W0, b0,
                     lng, lnb, proj, ad, exp8, W2):
    e = ed.shape[0]
    grid = (e // EB,)
    full = lambda shape: pl.BlockSpec(shape, lambda i: (0,) * len(shape))
    return pl.pallas_call(
        _edge_kernel,
        grid=grid,
        in_specs=[
            pl.BlockSpec((EB, EDGE), lambda i: (i, 0)),
            pl.BlockSpec((EB, L2 * C), lambda i: (i, 0)),
            pl.BlockSpec((EB, L2 * C), lambda i: (i, 0)),
            full((EDGE, 64)), full((1, 64)), full((64, H)), full((1, H)),
            full((3, 2 * C, H)), full((2 * C, EXTRA)), full((1, EXTRA)),
            full((1, HEADS * A)), full((1, HEADS * A)),
            full((HEADS * A, HEADS * A)), full((HEADS * A, HEADS)),
            full((HEADS, HEADS * V)), full((3, H, HEADS * V)),
        ],
        out_specs=pl.BlockSpec((EB, ROW), lambda i: (i, 0)),
        out_shape=jax.ShapeDtypeStruct((e, ROW), jnp.float32),
    )(ed, gs, gd, Wr1, br1, Wr2, br2, W1, W0, b0, lng, lnb, proj, ad, exp8, W2)


def _run_node_kernel(node, Wp, bp, exp8):
    n = node.shape[0]
    grid = (n // NB,)
    full = lambda shape: pl.BlockSpec(shape, lambda i: (0,) * len(shape))
    return pl.pallas_call(
        _node_kernel,
        grid=grid,
        in_specs=[
            pl.BlockSpec((NB, ROW), lambda i: (i, 0)),
            full((3, HEADS * V, OUT)), full((1, OUT)), full((HEADS, HEADS * V)),
        ],
        out_specs=pl.BlockSpec((NB, L2 * OUT), lambda i: (i, 0)),
        out_shape=jax.ShapeDtypeStruct((n, L2 * OUT), jnp.float32),
    )(node, Wp, bp, exp8)


def kernel(x_src, x_dst, edge_distance, edge_src, edge_dst, Wr1, br1, Wr2, br2,
           W1, W0, b0, ln_g, ln_b, alpha_dot, W2, Wp, bp):
    n = x_src.shape[0]
    e = edge_src.shape[0]
    f32 = jnp.float32

    lng = jnp.tile(ln_g, HEADS).reshape(1, HEADS * A)
    lnb = jnp.tile(ln_b, HEADS).reshape(1, HEADS * A)
    eyeh = jnp.eye(HEADS, dtype=f32)
    proj = jnp.kron(eyeh, jnp.ones((A, A), f32)) / A
    ad = (eyeh[:, None, :] * alpha_dot[:, :, None]).reshape(HEADS * A, HEADS)
    exp8 = jnp.kron(eyeh, jnp.ones((1, V), f32))

    xs2 = x_src.reshape(n, L2 * C)
    xd2 = x_dst.reshape(n, L2 * C)

    gs = xs2[edge_src]
    gd = xd2[edge_dst]

    val_s = _run_edge_kernel(
        edge_distance, gs, gd, Wr1, br1.reshape(1, -1), Wr2, br2.reshape(1, -1),
        W1, W0, b0.reshape(1, -1), lng, lnb, proj, ad, exp8, W2)

    node = jax.ops.segment_sum(val_s, edge_dst, num_segments=N_PAD)

    out = _run_node_kernel(node, Wp, bp.reshape(1, -1), exp8)
    return out[:n].reshape(n, L2, OUT)


# speedup vs baseline: 8.4122x; 1.2109x over previous
import functools

import jax
import jax.numpy as jnp
import numpy as np
from jax import lax
from jax.experimental import pallas as pl
from jax.experimental.pallas import tpu as pltpu
from jax.experimental.pallas import tpu_sc as plsc

N = 10000
E = 160000
L2 = 9
C = 32
H = 32
HEADS = 8
A = 16
V = 8
OUT = 32
EDGE = 64
EXTRA = HEADS * A + H
L_OF_PY = (0, 1, 1, 1, 2, 2, 2, 2, 2)

EB = 1280
NB = 1024
ROW = 640
N_PAD = 10240

SC_NC = 2
SC_NS = 16
SC_NW = SC_NC * SC_NS

GCH = 64
DG = 384


def _sc_gather_make(n, e, d):
    erows = e // GCH
    base_rows = ((erows + SC_NW - 1) // SC_NW + 7) // 8 * 8
    pad_rows = base_rows * SC_NW
    assert base_rows % 2 == 0 and (erows - base_rows * (SC_NW - 1)) % 2 == 0
    mesh = plsc.VectorSubcoreMesh(core_axis_name="c", subcore_axis_name="s")

    @functools.partial(
        pl.kernel,
        out_type=[jax.ShapeDtypeStruct((e, d), jnp.float32),
                  jax.ShapeDtypeStruct((e, d), jnp.float32)],
        mesh=mesh,
        scratch_types=[
            pltpu.VMEM((base_rows, GCH), jnp.int32),
            pltpu.VMEM((base_rows, GCH), jnp.int32),
            pltpu.VMEM((2, GCH, d), jnp.float32),
            pltpu.VMEM((2, GCH, d), jnp.float32),
            pltpu.SemaphoreType.DMA,
            pltpu.SemaphoreType.DMA,
            pltpu.SemaphoreType.DMA,
            pltpu.SemaphoreType.DMA,
        ],
    )
    def gk(xs_hbm, xd_hbm, es_hbm, ed_hbm, gs_hbm, gd_hbm,
           idxs, idxd, bs, bd, semg0, semg1, semw0, semw1):
        wid = lax.axis_index("s") * SC_NC + lax.axis_index("c")
        wrow = wid * base_rows
        nrows = jnp.minimum(base_rows, erows - wrow)
        pltpu.sync_copy(es_hbm.at[pl.ds(wrow, base_rows)], idxs)
        pltpu.sync_copy(ed_hbm.at[pl.ds(wrow, base_rows)], idxd)

        semg = (semg0, semg1)
        semw = (semw0, semw1)

        def gather(j, b):
            cs = pltpu.async_copy(xs_hbm.at[idxs.at[j]], bs.at[b], semg[b])
            cd = pltpu.async_copy(xd_hbm.at[idxd.at[j]], bd.at[b], semg[b])
            return cs, cd

        def wait_g(b):
            pltpu.make_async_copy(xs_hbm.at[idxs.at[0]], bs.at[b], semg[b]).wait()
            pltpu.make_async_copy(xd_hbm.at[idxd.at[0]], bd.at[b], semg[b]).wait()

        def writeback(j, b):
            ebase = (wrow + j) * GCH
            pltpu.async_copy(bs.at[b], gs_hbm.at[pl.ds(ebase, GCH)], semw[b])
            pltpu.async_copy(bd.at[b], gd_hbm.at[pl.ds(ebase, GCH)], semw[b])

        def wait_w(b):
            ebase = wrow * GCH
            pltpu.make_async_copy(bs.at[b], gs_hbm.at[pl.ds(ebase, GCH)], semw[b]).wait()
            pltpu.make_async_copy(bd.at[b], gd_hbm.at[pl.ds(ebase, GCH)], semw[b]).wait()

        gather(0, 0)
        gather(1, 1)

        def body(jp, _):
            wait_g(0)
            writeback(jp, 0)
            wait_g(1)
            writeback(jp + 1, 1)
            wait_w(0)

            @pl.when(jp + 2 < nrows)
            def _():
                gather(jp + 2, 0)
            wait_w(1)

            @pl.when(jp + 3 < nrows)
            def _():
                gather(jp + 3, 1)
            return 0

        lax.fori_loop(0, nrows // 2, lambda i, c: body(i * 2, c), 0, unroll=False)

    return gk


def _edge_kernel(ed_ref, gs_ref, gd_ref, wr1_ref, br1_ref, wr2_ref, br2_ref,
                 w1_ref, w0_ref, b0_ref, lng_ref, lnb_ref, proj_ref, ad_ref,
                 exp8_ref, w2_ref, out_ref):
    f32 = jnp.float32
    dot = functools.partial(jnp.dot, preferred_element_type=f32)

    ed = ed_ref[...]
    rad = dot(ed, wr1_ref[...]) + br1_ref[...]
    rad = rad * jax.nn.sigmoid(rad)
    gate = dot(rad, wr2_ref[...]) + br2_ref[...]

    gs0 = gs_ref[:, 0:C]
    gd0 = gd_ref[:, 0:C]
    extra = dot(gs0, w0_ref[0:C, :]) + dot(gd0, w0_ref[C:2 * C, :]) + b0_ref[...]
    gating = extra[:, HEADS * A:]
    sg = jax.nn.sigmoid(gating)
    scal = gating * sg

    xa = extra[:, :HEADS * A]
    proj = proj_ref[...]
    mu = dot(xa, proj)
    cent = xa - mu
    var = dot(cent * cent, proj)
    xan = cent * jax.lax.rsqrt(var + 1e-5) * lng_ref[...] + lnb_ref[...]
    xsl = xan * jax.nn.sigmoid(xan) * 0.8 + 0.2 * xan
    alpha = dot(xsl, ad_ref[...])
    s = jnp.exp(alpha)
    s64 = dot(s, exp8_ref[...])

    for i in range(L2):
        l = L_OF_PY[i]
        if i == 0:
            hid = scal
        else:
            gsi = gs_ref[:, i * C:(i + 1) * C]
            gdi = gd_ref[:, i * C:(i + 1) * C]
            hid = (dot(gsi, w1_ref[l, 0:C, :]) + dot(gdi, w1_ref[l, C:2 * C, :]))
            hid = hid * gate * sg
        val = dot(hid, w2_ref[l])
        out_ref[:, i * 64:(i + 1) * 64] = val * s64
    out_ref[:, 576:584] = s
    out_ref[:, 584:ROW] = jnp.zeros((ed.shape[0], ROW - 584), f32)


def _node_kernel(node_ref, wp_ref, bp_ref, exp8_ref, out_ref):
    f32 = jnp.float32
    dot = functools.partial(jnp.dot, preferred_element_type=f32)
    asum = node_ref[:, 576:584]
    inv = 1.0 / (asum + 1e-16)
    inv64 = dot(inv, exp8_ref[...])
    for i in range(L2):
        l = L_OF_PY[i]
        acc = node_ref[:, i * 64:(i + 1) * 64] * inv64
        o = dot(acc, wp_ref[l])
        if i == 0:
            o = o + bp_ref[...]
        out_ref[:, i * OUT:(i + 1) * OUT] = o


def _run_edge_kernel(ed, gs, gd, Wr1, br1, Wr2, br2, W1, W0, b0,
                     lng, lnb, proj, ad, exp8, W2):
    e = ed.shape[0]
    grid = (e // EB,)
    full = lambda shape: pl.BlockSpec(shape, lambda i: (0,) * len(shape))
    return pl.pallas_call(
        _edge_kernel,
        grid=grid,
        in_specs=[
            pl.BlockSpec((EB, EDGE), lambda i: (i, 0)),
            pl.BlockSpec((EB, DG), lambda i: (i, 0)),
            pl.BlockSpec((EB, DG), lambda i: (i, 0)),
            full((EDGE, 64)), full((1, 64)), full((64, H)), full((1, H)),
            full((3, 2 * C, H)), full((2 * C, EXTRA)), full((1, EXTRA)),
            full((1, HEADS * A)), full((1, HEADS * A)),
            full((HEADS * A, HEADS * A)), full((HEADS * A, HEADS)),
            full((HEADS, HEADS * V)), full((3, H, HEADS * V)),
        ],
        out_specs=pl.BlockSpec((EB, ROW), lambda i: (i, 0)),
        out_shape=jax.ShapeDtypeStruct((e, ROW), jnp.float32),
    )(ed, gs, gd, Wr1, br1, Wr2, br2, W1, W0, b0, lng, lnb, proj, ad, exp8, W2)


def _run_node_kernel(node, Wp, bp, exp8):
    n = node.shape[0]
    grid = (n // NB,)
    full = lambda shape: pl.BlockSpec(shape, lambda i: (0,) * len(shape))
    return pl.pallas_call(
        _node_kernel,
        grid=grid,
        in_specs=[
            pl.BlockSpec((NB, ROW), lambda i: (i, 0)),
            full((3, HEADS * V, OUT)), full((1, OUT)), full((HEADS, HEADS * V)),
        ],
        out_specs=pl.BlockSpec((NB, L2 * OUT), lambda i: (i, 0)),
        out_shape=jax.ShapeDtypeStruct((n, L2 * OUT), jnp.float32),
    )(node, Wp, bp, exp8)


def kernel(x_src, x_dst, edge_distance, edge_src, edge_dst, Wr1, br1, Wr2, br2,
           W1, W0, b0, ln_g, ln_b, alpha_dot, W2, Wp, bp):
    n = x_src.shape[0]
    e = edge_src.shape[0]
    f32 = jnp.float32

    lng = jnp.tile(ln_g, HEADS).reshape(1, HEADS * A)
    lnb = jnp.tile(ln_b, HEADS).reshape(1, HEADS * A)
    eyeh = jnp.eye(HEADS, dtype=f32)
    proj = jnp.kron(eyeh, jnp.ones((A, A), f32)) / A
    ad = (eyeh[:, None, :] * alpha_dot[:, :, None]).reshape(HEADS * A, HEADS)
    exp8 = jnp.kron(eyeh, jnp.ones((1, V), f32))

    xs2 = jnp.pad(x_src.reshape(n, L2 * C), ((0, 0), (0, DG - L2 * C)))
    xd2 = jnp.pad(x_dst.reshape(n, L2 * C), ((0, 0), (0, DG - L2 * C)))

    erows = e // GCH
    g_base_rows = ((erows + SC_NW - 1) // SC_NW + 7) // 8 * 8
    g_pad = g_base_rows * SC_NW - erows
    es2 = jnp.pad(edge_src.reshape(erows, GCH), ((0, g_pad), (0, 0)))
    ed2 = jnp.pad(edge_dst.reshape(erows, GCH), ((0, g_pad), (0, 0)))
    gs, gd = _sc_gather_make(n, e, DG)(xs2, xd2, es2, ed2)

    val_s = _run_edge_kernel(
        edge_distance, gs, gd, Wr1, br1.reshape(1, -1), Wr2, br2.reshape(1, -1),
        W1, W0, b0.reshape(1, -1), lng, lnb, proj, ad, exp8, W2)

    node = jax.ops.segment_sum(val_s, edge_dst, num_segments=N_PAD)

    out = _run_node_kernel(node, Wp, bp.reshape(1, -1), exp8)
    return out[:n].reshape(n, L2, OUT)


# speedup vs baseline: 12.7889x; 1.5203x over previous
import functools

import jax
import jax.numpy as jnp
import numpy as np
from jax import lax
from jax.experimental import pallas as pl
from jax.experimental.pallas import tpu as pltpu
from jax.experimental.pallas import tpu_sc as plsc

N = 10000
E = 160000
L2 = 9
C = 32
H = 32
HEADS = 8
A = 16
V = 8
OUT = 32
EDGE = 64
EXTRA = HEADS * A + H
L_OF_PY = (0, 1, 1, 1, 2, 2, 2, 2, 2)

EB = 1280
NB = 1024
SCW = 128
NCHUNK = 5
N_PAD = 10240

SC_NC = 2
SC_NS = 16
SC_NW = SC_NC * SC_NS

GCH = 64
DG = 384


def _sc_gather_make(n, e, d):
    erows = e // GCH
    base_rows = ((erows + SC_NW - 1) // SC_NW + 7) // 8 * 8
    pad_rows = base_rows * SC_NW
    assert base_rows % 2 == 0 and (erows - base_rows * (SC_NW - 1)) % 2 == 0
    mesh = plsc.VectorSubcoreMesh(core_axis_name="c", subcore_axis_name="s")

    @functools.partial(
        pl.kernel,
        out_type=[jax.ShapeDtypeStruct((e, d), jnp.float32),
                  jax.ShapeDtypeStruct((e, d), jnp.float32)],
        mesh=mesh,
        scratch_types=[
            pltpu.VMEM((base_rows, GCH), jnp.int32),
            pltpu.VMEM((base_rows, GCH), jnp.int32),
            pltpu.VMEM((2, GCH, d), jnp.float32),
            pltpu.VMEM((2, GCH, d), jnp.float32),
            pltpu.SemaphoreType.DMA,
            pltpu.SemaphoreType.DMA,
            pltpu.SemaphoreType.DMA,
            pltpu.SemaphoreType.DMA,
        ],
    )
    def gk(xs_hbm, xd_hbm, es_hbm, ed_hbm, gs_hbm, gd_hbm,
           idxs, idxd, bs, bd, semg0, semg1, semw0, semw1):
        wid = lax.axis_index("s") * SC_NC + lax.axis_index("c")
        wrow = wid * base_rows
        nrows = jnp.minimum(base_rows, erows - wrow)
        pltpu.sync_copy(es_hbm.at[pl.ds(wrow, base_rows)], idxs)
        pltpu.sync_copy(ed_hbm.at[pl.ds(wrow, base_rows)], idxd)

        semg = (semg0, semg1)
        semw = (semw0, semw1)

        def gather(j, b):
            cs = pltpu.async_copy(xs_hbm.at[idxs.at[j]], bs.at[b], semg[b])
            cd = pltpu.async_copy(xd_hbm.at[idxd.at[j]], bd.at[b], semg[b])
            return cs, cd

        def wait_g(b):
            pltpu.make_async_copy(xs_hbm.at[idxs.at[0]], bs.at[b], semg[b]).wait()
            pltpu.make_async_copy(xd_hbm.at[idxd.at[0]], bd.at[b], semg[b]).wait()

        def writeback(j, b):
            ebase = (wrow + j) * GCH
            pltpu.async_copy(bs.at[b], gs_hbm.at[pl.ds(ebase, GCH)], semw[b])
            pltpu.async_copy(bd.at[b], gd_hbm.at[pl.ds(ebase, GCH)], semw[b])

        def wait_w(b):
            ebase = wrow * GCH
            pltpu.make_async_copy(bs.at[b], gs_hbm.at[pl.ds(ebase, GCH)], semw[b]).wait()
            pltpu.make_async_copy(bd.at[b], gd_hbm.at[pl.ds(ebase, GCH)], semw[b]).wait()

        gather(0, 0)
        gather(1, 1)

        def body(jp, _):
            wait_g(0)
            writeback(jp, 0)
            wait_g(1)
            writeback(jp + 1, 1)
            wait_w(0)

            @pl.when(jp + 2 < nrows)
            def _():
                gather(jp + 2, 0)
            wait_w(1)

            @pl.when(jp + 3 < nrows)
            def _():
                gather(jp + 3, 1)
            return 0

        lax.fori_loop(0, nrows // 2, lambda i, c: body(i * 2, c), 0, unroll=False)

    return gk


SCH = 128


def _sc_scatter_make(e, n_pad):
    erows = e // SCH
    base_rows = ((erows + SC_NS - 1) // SC_NS + 7) // 8 * 8
    out_rows = n_pad // SC_NS
    mesh = plsc.VectorSubcoreMesh(core_axis_name="c", subcore_axis_name="s")

    @functools.partial(
        pl.kernel,
        out_type=[jax.ShapeDtypeStruct((n_pad, SCW), jnp.float32)] * NCHUNK,
        mesh=mesh,
        scratch_types=[
            pltpu.VMEM((base_rows, SCH), jnp.int32),
            pltpu.VMEM((2, SCH, SCW), jnp.float32),
            pltpu.VMEM_SHARED((n_pad, SCW), jnp.float32),
            pltpu.SemaphoreType.DMA,
            pltpu.SemaphoreType.DMA,
            pltpu.SemaphoreType.DMA,
            pltpu.SemaphoreType.DMA,
        ],
    )
    def sk(v0, v1, v2, v3, v4, ed_hbm, z_hbm, o0, o1, o2, o3, o4,
           idx, rows, acc, r0, r1, s0, s1):
        c = lax.axis_index("c")
        s = lax.axis_index("s")
        trow = s * base_rows
        nrows = jnp.minimum(base_rows, erows - trow)
        pltpu.sync_copy(ed_hbm.at[pl.ds(trow, base_rows)], idx)
        orow = s * out_rows
        semr = (r0, r1)
        sems = (s0, s1)

        def do_chunk(v_hbm, o_hbm):
            pltpu.sync_copy(z_hbm.at[pl.ds(orow, out_rows)],
                            acc.at[pl.ds(orow, out_rows)])
            plsc.subcore_barrier()

            def load(j, b):
                pltpu.async_copy(v_hbm.at[pl.ds((trow + j) * SCH, SCH)],
                                 rows.at[b], semr[b])

            def wait_r(b):
                pltpu.make_async_copy(v_hbm.at[pl.ds(0, SCH)], rows.at[b],
                                      semr[b]).wait()

            def scat(j, b):
                pltpu.async_copy(rows.at[b], acc.at[idx.at[j]], sems[b],
                                 add=True)

            def wait_s(b):
                pltpu.make_async_copy(rows.at[b], acc.at[idx.at[0]],
                                      sems[b]).wait()

            load(0, 0)
            load(1, 1)

            def body(jp, _):
                wait_r(0)
                scat(jp, 0)
                wait_r(1)
                scat(jp + 1, 1)
                wait_s(0)

                @pl.when(jp + 2 < nrows)
                def _():
                    load(jp + 2, 0)
                wait_s(1)

                @pl.when(jp + 3 < nrows)
                def _():
                    load(jp + 3, 1)
                return 0

            lax.fori_loop(0, nrows // 2, lambda i, cr: body(i * 2, cr), 0,
                          unroll=False)
            plsc.subcore_barrier()
            pltpu.sync_copy(acc.at[pl.ds(orow, out_rows)],
                            o_hbm.at[pl.ds(orow, out_rows)])
            plsc.subcore_barrier()

        @pl.when(c == 0)
        def _():
            do_chunk(v0, o0)
            do_chunk(v1, o1)
            do_chunk(v2, o2)

        @pl.when(c == 1)
        def _():
            do_chunk(v3, o3)
            do_chunk(v4, o4)

    return sk


def _edge_kernel(ed_ref, gs_ref, gd_ref, wr1_ref, br1_ref, wr2_ref, br2_ref,
                 w1_ref, w0_ref, b0_ref, lng_ref, lnb_ref, proj_ref, ad_ref,
                 exp8_ref, w2_ref, out0_ref, out1_ref, out2_ref, out3_ref,
                 out4_ref):
    f32 = jnp.float32
    dot = functools.partial(jnp.dot, preferred_element_type=f32)

    ed = ed_ref[...]
    rad = dot(ed, wr1_ref[...]) + br1_ref[...]
    rad = rad * jax.nn.sigmoid(rad)
    gate = dot(rad, wr2_ref[...]) + br2_ref[...]

    gs0 = gs_ref[:, 0:C]
    gd0 = gd_ref[:, 0:C]
    extra = dot(gs0, w0_ref[0:C, :]) + dot(gd0, w0_ref[C:2 * C, :]) + b0_ref[...]
    gating = extra[:, HEADS * A:]
    sg = jax.nn.sigmoid(gating)
    scal = gating * sg

    xa = extra[:, :HEADS * A]
    proj = proj_ref[...]
    mu = dot(xa, proj)
    cent = xa - mu
    var = dot(cent * cent, proj)
    xan = cent * jax.lax.rsqrt(var + 1e-5) * lng_ref[...] + lnb_ref[...]
    xsl = xan * jax.nn.sigmoid(xan) * 0.8 + 0.2 * xan
    alpha = dot(xsl, ad_ref[...])
    s = jnp.exp(alpha)
    s64 = dot(s, exp8_ref[...])

    outs = (out0_ref, out1_ref, out2_ref, out3_ref, out4_ref)
    for i in range(L2):
        l = L_OF_PY[i]
        if i == 0:
            hid = scal
        else:
            gsi = gs_ref[:, i * C:(i + 1) * C]
            gdi = gd_ref[:, i * C:(i + 1) * C]
            hid = (dot(gsi, w1_ref[l, 0:C, :]) + dot(gdi, w1_ref[l, C:2 * C, :]))
            hid = hid * gate * sg
        val = dot(hid, w2_ref[l])
        outs[i // 2][:, (i % 2) * 64:(i % 2) * 64 + 64] = val * s64
    out4_ref[:, 64:72] = s
    out4_ref[:, 72:SCW] = jnp.zeros((ed.shape[0], SCW - 72), f32)


def _node_kernel(n0_ref, n1_ref, n2_ref, n3_ref, n4_ref, wp_ref, bp_ref,
                 exp8_ref, out_ref):
    f32 = jnp.float32
    dot = functools.partial(jnp.dot, preferred_element_type=f32)
    ns = (n0_ref, n1_ref, n2_ref, n3_ref, n4_ref)
    asum = n4_ref[:, 64:72]
    inv = 1.0 / (asum + 1e-16)
    inv64 = dot(inv, exp8_ref[...])
    for i in range(L2):
        l = L_OF_PY[i]
        acc = ns[i // 2][:, (i % 2) * 64:(i % 2) * 64 + 64] * inv64
        o = dot(acc, wp_ref[l])
        if i == 0:
            o = o + bp_ref[...]
        out_ref[:, i * OUT:(i + 1) * OUT] = o


def _run_edge_kernel(ed, gs, gd, Wr1, br1, Wr2, br2, W1, W0, b0,
                     lng, lnb, proj, ad, exp8, W2):
    e = ed.shape[0]
    grid = (e // EB,)
    full = lambda shape: pl.BlockSpec(shape, lambda i: (0,) * len(shape))
    return pl.pallas_call(
        _edge_kernel,
        grid=grid,
        in_specs=[
            pl.BlockSpec((EB, EDGE), lambda i: (i, 0)),
            pl.BlockSpec((EB, DG), lambda i: (i, 0)),
            pl.BlockSpec((EB, DG), lambda i: (i, 0)),
            full((EDGE, 64)), full((1, 64)), full((64, H)), full((1, H)),
            full((3, 2 * C, H)), full((2 * C, EXTRA)), full((1, EXTRA)),
            full((1, HEADS * A)), full((1, HEADS * A)),
            full((HEADS * A, HEADS * A)), full((HEADS * A, HEADS)),
            full((HEADS, HEADS * V)), full((3, H, HEADS * V)),
        ],
        out_specs=[pl.BlockSpec((EB, SCW), lambda i: (i, 0))] * NCHUNK,
        out_shape=[jax.ShapeDtypeStruct((e, SCW), jnp.float32)] * NCHUNK,
    )(ed, gs, gd, Wr1, br1, Wr2, br2, W1, W0, b0, lng, lnb, proj, ad, exp8, W2)


def _run_node_kernel(nodes, Wp, bp, exp8):
    n = nodes[0].shape[0]
    grid = (n // NB,)
    full = lambda shape: pl.BlockSpec(shape, lambda i: (0,) * len(shape))
    return pl.pallas_call(
        _node_kernel,
        grid=grid,
        in_specs=[pl.BlockSpec((NB, SCW), lambda i: (i, 0))] * NCHUNK + [
            full((3, HEADS * V, OUT)), full((1, OUT)), full((HEADS, HEADS * V)),
        ],
        out_specs=pl.BlockSpec((NB, L2 * OUT), lambda i: (i, 0)),
        out_shape=jax.ShapeDtypeStruct((n, L2 * OUT), jnp.float32),
    )(*nodes, Wp, bp, exp8)


def kernel(x_src, x_dst, edge_distance, edge_src, edge_dst, Wr1, br1, Wr2, br2,
           W1, W0, b0, ln_g, ln_b, alpha_dot, W2, Wp, bp):
    n = x_src.shape[0]
    e = edge_src.shape[0]
    f32 = jnp.float32

    lng = jnp.tile(ln_g, HEADS).reshape(1, HEADS * A)
    lnb = jnp.tile(ln_b, HEADS).reshape(1, HEADS * A)
    eyeh = jnp.eye(HEADS, dtype=f32)
    proj = jnp.kron(eyeh, jnp.ones((A, A), f32)) / A
    ad = (eyeh[:, None, :] * alpha_dot[:, :, None]).reshape(HEADS * A, HEADS)
    exp8 = jnp.kron(eyeh, jnp.ones((1, V), f32))

    xs2 = jnp.pad(x_src.reshape(n, L2 * C), ((0, 0), (0, DG - L2 * C)))
    xd2 = jnp.pad(x_dst.reshape(n, L2 * C), ((0, 0), (0, DG - L2 * C)))

    erows = e // GCH
    g_base_rows = ((erows + SC_NW - 1) // SC_NW + 7) // 8 * 8
    g_pad = g_base_rows * SC_NW - erows
    es2 = jnp.pad(edge_src.reshape(erows, GCH), ((0, g_pad), (0, 0)))
    ed2 = jnp.pad(edge_dst.reshape(erows, GCH), ((0, g_pad), (0, 0)))
    gs, gd = _sc_gather_make(n, e, DG)(xs2, xd2, es2, ed2)

    vals = _run_edge_kernel(
        edge_distance, gs, gd, Wr1, br1.reshape(1, -1), Wr2, br2.reshape(1, -1),
        W1, W0, b0.reshape(1, -1), lng, lnb, proj, ad, exp8, W2)

    srows = e // SCH
    s_base_rows = ((srows + SC_NS - 1) // SC_NS + 7) // 8 * 8
    s_pad = s_base_rows * SC_NS - srows
    eds2 = jnp.pad(edge_dst.reshape(srows, SCH), ((0, s_pad), (0, 0)))
    z = jnp.zeros((N_PAD, SCW), f32)
    nodes = _sc_scatter_make(e, N_PAD)(*vals, eds2, z)

    out = _run_node_kernel(nodes, Wp, bp.reshape(1, -1), exp8)
    return out[:n].reshape(n, L2, OUT)


# speedup vs baseline: 16.3571x; 1.2790x over previous
import functools

import jax
import jax.numpy as jnp
import numpy as np
from jax import lax
from jax.experimental import pallas as pl
from jax.experimental.pallas import tpu as pltpu
from jax.experimental.pallas import tpu_sc as plsc

N = 10000
E = 160000
L2 = 9
C = 32
H = 32
HEADS = 8
A = 16
V = 8
OUT = 32
EDGE = 64
EXTRA = HEADS * A + H
L_OF_PY = (0, 1, 1, 1, 2, 2, 2, 2, 2)

EB = 1280
NB = 1024
SCW = 128
NCHUNK = 5
N_PAD = 10240

SC_NC = 2
SC_NS = 16
SC_NW = SC_NC * SC_NS

GCH = 64
DG = 384


def _sc_gather_make(n, e, d):
    erows = e // GCH
    base_rows = ((erows + SC_NW - 1) // SC_NW + 7) // 8 * 8
    pad_rows = base_rows * SC_NW
    assert base_rows % 2 == 0 and (erows - base_rows * (SC_NW - 1)) % 2 == 0
    mesh = plsc.VectorSubcoreMesh(core_axis_name="c", subcore_axis_name="s")

    @functools.partial(
        pl.kernel,
        out_type=[jax.ShapeDtypeStruct((e, d), jnp.float32),
                  jax.ShapeDtypeStruct((e, d), jnp.float32)],
        mesh=mesh,
        scratch_types=[
            pltpu.VMEM((base_rows, GCH), jnp.int32),
            pltpu.VMEM((base_rows, GCH), jnp.int32),
            pltpu.VMEM((2, GCH, d), jnp.float32),
            pltpu.VMEM((2, GCH, d), jnp.float32),
            pltpu.SemaphoreType.DMA,
            pltpu.SemaphoreType.DMA,
            pltpu.SemaphoreType.DMA,
            pltpu.SemaphoreType.DMA,
        ],
    )
    def gk(xs_hbm, xd_hbm, es_hbm, ed_hbm, gs_hbm, gd_hbm,
           idxs, idxd, bs, bd, semg0, semg1, semw0, semw1):
        wid = lax.axis_index("s") * SC_NC + lax.axis_index("c")
        wrow = wid * base_rows
        nrows = jnp.minimum(base_rows, erows - wrow)
        pltpu.sync_copy(es_hbm.at[pl.ds(wrow, base_rows)], idxs)
        pltpu.sync_copy(ed_hbm.at[pl.ds(wrow, base_rows)], idxd)

        semg = (semg0, semg1)
        semw = (semw0, semw1)

        def gather(j, b):
            cs = pltpu.async_copy(xs_hbm.at[idxs.at[j]], bs.at[b], semg[b])
            cd = pltpu.async_copy(xd_hbm.at[idxd.at[j]], bd.at[b], semg[b])
            return cs, cd

        def wait_g(b):
            pltpu.make_async_copy(xs_hbm.at[idxs.at[0]], bs.at[b], semg[b]).wait()
            pltpu.make_async_copy(xd_hbm.at[idxd.at[0]], bd.at[b], semg[b]).wait()

        def writeback(j, b):
            ebase = (wrow + j) * GCH
            pltpu.async_copy(bs.at[b], gs_hbm.at[pl.ds(ebase, GCH)], semw[b])
            pltpu.async_copy(bd.at[b], gd_hbm.at[pl.ds(ebase, GCH)], semw[b])

        def wait_w(b):
            ebase = wrow * GCH
            pltpu.make_async_copy(bs.at[b], gs_hbm.at[pl.ds(ebase, GCH)], semw[b]).wait()
            pltpu.make_async_copy(bd.at[b], gd_hbm.at[pl.ds(ebase, GCH)], semw[b]).wait()

        gather(0, 0)
        gather(1, 1)

        def body(jp, _):
            wait_g(0)
            writeback(jp, 0)
            wait_g(1)
            writeback(jp + 1, 1)
            wait_w(0)

            @pl.when(jp + 2 < nrows)
            def _():
                gather(jp + 2, 0)
            wait_w(1)

            @pl.when(jp + 3 < nrows)
            def _():
                gather(jp + 3, 1)
            return 0

        lax.fori_loop(0, nrows // 2, lambda i, c: body(i * 2, c), 0, unroll=False)

    return gk


SCH = 128


def _sc_scatter_make(e, n_pad):
    erows = e // SCH
    base_rows = ((erows + SC_NS - 1) // SC_NS + 7) // 8 * 8
    half = erows // 2 // 8 * 8
    half_bp = ((max(half, erows - half) + SC_NS - 1) // SC_NS + 7) // 8 * 8
    out_rows = n_pad // SC_NS
    mesh = plsc.VectorSubcoreMesh(core_axis_name="c", subcore_axis_name="s")

    @functools.partial(
        pl.kernel,
        out_type=[jax.ShapeDtypeStruct((n_pad, SCW), jnp.float32)] * (NCHUNK + 1),
        mesh=mesh,
        scratch_types=[
            pltpu.VMEM((base_rows, SCH), jnp.int32),
            pltpu.VMEM((half_bp, SCH), jnp.int32),
            pltpu.VMEM((2, SCH, SCW), jnp.float32),
            pltpu.VMEM_SHARED((n_pad, SCW), jnp.float32),
            pltpu.SemaphoreType.DMA,
            pltpu.SemaphoreType.DMA,
            pltpu.SemaphoreType.DMA,
            pltpu.SemaphoreType.DMA,
        ],
    )
    def sk(v0, v1, v2, v3, v4, ed_hbm, z_hbm, o0, o1, o2a, o2b, o3, o4,
           idxf, idxh, rows, acc, r0, r1, s0, s1):
        c = lax.axis_index("c")
        s = lax.axis_index("s")
        pltpu.sync_copy(ed_hbm.at[pl.ds(s * base_rows, base_rows)], idxf)
        orow = s * out_rows
        semr = (r0, r1)
        sems = (s0, s1)

        def do_chunk(v_hbm, o_hbm, idx, tstart, nrows):
            pltpu.sync_copy(z_hbm.at[pl.ds(orow, out_rows)],
                            acc.at[pl.ds(orow, out_rows)])
            plsc.subcore_barrier()

            def load(j, b):
                pltpu.async_copy(v_hbm.at[pl.ds((tstart + j) * SCH, SCH)],
                                 rows.at[b], semr[b])

            def wait_r(b):
                pltpu.make_async_copy(v_hbm.at[pl.ds(0, SCH)], rows.at[b],
                                      semr[b]).wait()

            def scat(j, b):
                pltpu.async_copy(rows.at[b], acc.at[idx.at[j]], sems[b],
                                 add=True)

            def wait_s(b):
                pltpu.make_async_copy(rows.at[b], acc.at[idx.at[0]],
                                      sems[b]).wait()

            @pl.when(nrows > 0)
            def _():
                load(0, 0)
                load(1, 1)

            def body(jp, _):
                wait_r(0)
                scat(jp, 0)
                wait_r(1)
                scat(jp + 1, 1)
                wait_s(0)

                @pl.when(jp + 2 < nrows)
                def _():
                    load(jp + 2, 0)
                wait_s(1)

                @pl.when(jp + 3 < nrows)
                def _():
                    load(jp + 3, 1)
                return 0

            lax.fori_loop(0, nrows // 2, lambda i, cr: body(i * 2, cr), 0,
                          unroll=False)
            plsc.subcore_barrier()
            pltpu.sync_copy(acc.at[pl.ds(orow, out_rows)],
                            o_hbm.at[pl.ds(orow, out_rows)])
            plsc.subcore_barrier()

        def full(v_hbm, o_hbm):
            nrows = jnp.minimum(base_rows, erows - s * base_rows)
            do_chunk(v_hbm, o_hbm, idxf, s * base_rows, nrows)

        def partial(v_hbm, o_hbm, rlo, count):
            tstart = rlo + s * half_bp
            pltpu.sync_copy(ed_hbm.at[pl.ds(tstart, half_bp)], idxh)
            nrows = jnp.clip(count - s * half_bp, 0, half_bp)
            do_chunk(v_hbm, o_hbm, idxh, tstart, nrows)

        @pl.when(c == 0)
        def _():
            full(v0, o0)
            full(v1, o1)
            partial(v2, o2a, 0, half)

        @pl.when(c == 1)
        def _():
            full(v3, o3)
            full(v4, o4)
            partial(v2, o2b, half, erows - half)

    return sk


def _table_kernel(x_ref, wbd_ref, out_ref):
    f32 = jnp.float32
    x = x_ref[...]
    y = jnp.dot(x, wbd_ref[...], preferred_element_type=f32)
    out_ref[:, 0:L2 * C] = y
    out_ref[:, L2 * C:L2 * C + C] = x[:, 0:C]
    out_ref[:, L2 * C + C:DG] = jnp.zeros((x.shape[0], DG - L2 * C - C), f32)


def _edge_kernel(ed_ref, gs_ref, gd_ref, wr1_ref, br1_ref, wr2_ref, br2_ref,
                 w0_ref, b0_ref, lng_ref, lnb_ref, proj_ref, ad_ref,
                 g9_ref, w2bd_ref, expbig_ref, out0_ref, out1_ref, out2_ref,
                 out3_ref, out4_ref):
    f32 = jnp.float32
    dot = functools.partial(jnp.dot, preferred_element_type=f32)

    ed = ed_ref[...]
    rad = dot(ed, wr1_ref[...]) + br1_ref[...]
    rad = rad * jax.nn.sigmoid(rad)
    gate = dot(rad, wr2_ref[...]) + br2_ref[...]

    gs0 = gs_ref[:, L2 * C:L2 * C + C]
    gd0 = gd_ref[:, L2 * C:L2 * C + C]
    extra = dot(gs0, w0_ref[0:C, :]) + dot(gd0, w0_ref[C:2 * C, :]) + b0_ref[...]
    gating = extra[:, HEADS * A:]
    sg = jax.nn.sigmoid(gating)
    scal = gating * sg

    xa = extra[:, :HEADS * A]
    proj = proj_ref[...]
    mu = dot(xa, proj)
    cent = xa - mu
    var = dot(cent * cent, proj)
    xan = cent * jax.lax.rsqrt(var + 1e-5) * lng_ref[...] + lnb_ref[...]
    xsl = xan * jax.nn.sigmoid(xan) * 0.8 + 0.2 * xan
    alpha = dot(xsl, ad_ref[...])
    s = jnp.exp(alpha)

    g9 = g9_ref[...]
    mod = dot(gate * sg, g9)
    hidm = (gs_ref[:, 0:L2 * C] + gd_ref[:, 0:L2 * C]) * mod
    hid = jnp.concatenate([scal, hidm[:, C:L2 * C]], axis=1)
    val = dot(hid.astype(jnp.bfloat16), w2bd_ref[...])
    attn = val * dot(s, expbig_ref[...])

    out0_ref[...] = attn[:, 0:128]
    out1_ref[...] = attn[:, 128:256]
    out2_ref[...] = attn[:, 256:384]
    out3_ref[...] = attn[:, 384:512]
    out4_ref[:, 0:64] = attn[:, 512:576]
    out4_ref[:, 64:72] = s
    out4_ref[:, 72:SCW] = jnp.zeros((ed.shape[0], SCW - 72), f32)


def _node_kernel(n0_ref, n1_ref, n2a_ref, n2b_ref, n3_ref, n4_ref, wp_ref,
                 bp_ref, exp8_ref, out_ref):
    f32 = jnp.float32
    dot = functools.partial(jnp.dot, preferred_element_type=f32)
    n2 = n2a_ref[...] + n2b_ref[...]
    ns = (n0_ref[...], n1_ref[...], n2, n3_ref[...], n4_ref[...])
    asum = ns[4][:, 64:72]
    inv = 1.0 / (asum + 1e-16)
    inv64 = dot(inv, exp8_ref[...])
    for i in range(L2):
        l = L_OF_PY[i]
        acc = ns[i // 2][:, (i % 2) * 64:(i % 2) * 64 + 64] * inv64
        o = dot(acc, wp_ref[l])
        if i == 0:
            o = o + bp_ref[...]
        out_ref[:, i * OUT:(i + 1) * OUT] = o


def _run_table_kernel(x2, wbd):
    n = x2.shape[0]
    nb = 1000 if n % 1000 == 0 else n
    return pl.pallas_call(
        _table_kernel,
        grid=(n // nb,),
        in_specs=[
            pl.BlockSpec((nb, L2 * C), lambda i: (i, 0)),
            pl.BlockSpec((L2 * C, L2 * C), lambda i: (0, 0)),
        ],
        out_specs=pl.BlockSpec((nb, DG), lambda i: (i, 0)),
        out_shape=jax.ShapeDtypeStruct((n, DG), jnp.float32),
    )(x2, wbd)


def _run_edge_kernel(ed, gs, gd, Wr1, br1, Wr2, br2, W0, b0,
                     lng, lnb, proj, ad, g9, w2bd, expbig):
    e = ed.shape[0]
    grid = (e // EB,)
    full = lambda shape: pl.BlockSpec(shape, lambda i: (0,) * len(shape))
    return pl.pallas_call(
        _edge_kernel,
        grid=grid,
        in_specs=[
            pl.BlockSpec((EB, EDGE), lambda i: (i, 0)),
            pl.BlockSpec((EB, DG), lambda i: (i, 0)),
            pl.BlockSpec((EB, DG), lambda i: (i, 0)),
            full((EDGE, 64)), full((1, 64)), full((64, H)), full((1, H)),
            full((2 * C, EXTRA)), full((1, EXTRA)),
            full((1, HEADS * A)), full((1, HEADS * A)),
            full((HEADS * A, HEADS * A)), full((HEADS * A, HEADS)),
            full((H, L2 * C)), full((L2 * C, 576)), full((HEADS, 576)),
        ],
        out_specs=[pl.BlockSpec((EB, SCW), lambda i: (i, 0))] * NCHUNK,
        out_shape=[jax.ShapeDtypeStruct((e, SCW), jnp.float32)] * NCHUNK,
    )(ed, gs, gd, Wr1, br1, Wr2, br2, W0, b0, lng, lnb, proj, ad, g9, w2bd,
      expbig)


def _run_node_kernel(nodes, Wp, bp, exp8):
    n = nodes[0].shape[0]
    grid = (n // NB,)
    full = lambda shape: pl.BlockSpec(shape, lambda i: (0,) * len(shape))
    return pl.pallas_call(
        _node_kernel,
        grid=grid,
        in_specs=[pl.BlockSpec((NB, SCW), lambda i: (i, 0))] * (NCHUNK + 1) + [
            full((3, HEADS * V, OUT)), full((1, OUT)), full((HEADS, HEADS * V)),
        ],
        out_specs=pl.BlockSpec((NB, L2 * OUT), lambda i: (i, 0)),
        out_shape=jax.ShapeDtypeStruct((n, L2 * OUT), jnp.float32),
    )(*nodes, Wp, bp, exp8)


def kernel(x_src, x_dst, edge_distance, edge_src, edge_dst, Wr1, br1, Wr2, br2,
           W1, W0, b0, ln_g, ln_b, alpha_dot, W2, Wp, bp):
    n = x_src.shape[0]
    e = edge_src.shape[0]
    f32 = jnp.float32

    lng = jnp.tile(ln_g, HEADS).reshape(1, HEADS * A)
    lnb = jnp.tile(ln_b, HEADS).reshape(1, HEADS * A)
    eyeh = jnp.eye(HEADS, dtype=f32)
    proj = jnp.kron(eyeh, jnp.ones((A, A), f32)) / A
    ad = (eyeh[:, None, :] * alpha_dot[:, :, None]).reshape(HEADS * A, HEADS)
    exp8 = jnp.kron(eyeh, jnp.ones((1, V), f32))
    eyec = jnp.eye(C, dtype=f32)
    g9 = jnp.concatenate([eyec] * L2, axis=1)
    w1s_bd = jax.scipy.linalg.block_diag(*[W1[l, 0:C, :] for l in L_OF_PY])
    w1d_bd = jax.scipy.linalg.block_diag(*[W1[l, C:2 * C, :] for l in L_OF_PY])
    w2_bd = jax.scipy.linalg.block_diag(
        *[W2[l] for l in L_OF_PY]).astype(jnp.bfloat16)
    expbig = jnp.tile(exp8, (1, L2))

    ts = _run_table_kernel(x_src.reshape(n, L2 * C), w1s_bd)
    td = _run_table_kernel(x_dst.reshape(n, L2 * C), w1d_bd)

    erows = e // GCH
    g_base_rows = ((erows + SC_NW - 1) // SC_NW + 7) // 8 * 8
    g_pad = g_base_rows * SC_NW - erows
    es2 = jnp.pad(edge_src.reshape(erows, GCH), ((0, g_pad), (0, 0)))
    ed2 = jnp.pad(edge_dst.reshape(erows, GCH), ((0, g_pad), (0, 0)))
    gs, gd = _sc_gather_make(n, e, DG)(ts, td, es2, ed2)

    vals = _run_edge_kernel(
        edge_distance, gs, gd, Wr1, br1.reshape(1, -1), Wr2, br2.reshape(1, -1),
        W0, b0.reshape(1, -1), lng, lnb, proj, ad, g9, w2_bd, expbig)

    srows = e // SCH
    s_base_rows = ((srows + SC_NS - 1) // SC_NS + 7) // 8 * 8
    s_pad = s_base_rows * SC_NS - srows
    eds2 = jnp.pad(edge_dst.reshape(srows, SCH), ((0, s_pad), (0, 0)))
    z = jnp.zeros((N_PAD, SCW), f32)
    nodes = _sc_scatter_make(e, N_PAD)(*vals, eds2, z)

    out = _run_node_kernel(nodes, Wp, bp.reshape(1, -1), exp8)
    return out[:n].reshape(n, L2, OUT)


# speedup vs baseline: 18.0912x; 1.1060x over previous
import functools

import jax
import jax.numpy as jnp
import numpy as np
from jax import lax
from jax.experimental import pallas as pl
from jax.experimental.pallas import tpu as pltpu
from jax.experimental.pallas import tpu_sc as plsc

N = 10000
E = 160000
L2 = 9
C = 32
H = 32
HEADS = 8
A = 16
V = 8
OUT = 32
EDGE = 64
EXTRA = HEADS * A + H
L_OF_PY = (0, 1, 1, 1, 2, 2, 2, 2, 2)

EB = 1280
NB = 1024
SCW = 128
NCHUNK = 5
N_PAD = 10240

SC_NC = 2
SC_NS = 16
SC_NW = SC_NC * SC_NS

GCH = 64
DG = 384


def _sc_gather_make(n, e):
    erows = e // GCH
    base_rows = ((erows + SC_NW - 1) // SC_NW + 7) // 8 * 8
    assert base_rows % 2 == 0 and (erows - base_rows * (SC_NW - 1)) % 2 == 0
    mesh = plsc.VectorSubcoreMesh(core_axis_name="c", subcore_axis_name="s")

    @functools.partial(
        pl.kernel,
        out_type=[jax.ShapeDtypeStruct((e, 128), jnp.int32),
                  jax.ShapeDtypeStruct((e, 128), jnp.int32),
                  jax.ShapeDtypeStruct((e, SCW), jnp.float32),
                  jax.ShapeDtypeStruct((e, SCW), jnp.float32)],
        mesh=mesh,
        scratch_types=[
            pltpu.VMEM((base_rows, GCH), jnp.int32),
            pltpu.VMEM((base_rows, GCH), jnp.int32),
            pltpu.VMEM((2, GCH, 128), jnp.int32),
            pltpu.VMEM((2, GCH, 128), jnp.int32),
            pltpu.VMEM((2, GCH, SCW), jnp.float32),
            pltpu.VMEM((2, GCH, SCW), jnp.float32),
            pltpu.SemaphoreType.DMA,
            pltpu.SemaphoreType.DMA,
            pltpu.SemaphoreType.DMA,
            pltpu.SemaphoreType.DMA,
        ],
    )
    def gk(as_hbm, ad_hbm, bs_hbm, bd_hbm, es_hbm, ed_hbm,
           gas_hbm, gad_hbm, gbs_hbm, gbd_hbm,
           idxs, idxd, bas, bad, bbs, bbd, semg0, semg1, semw0, semw1):
        wid = lax.axis_index("s") * SC_NC + lax.axis_index("c")
        wrow = wid * base_rows
        nrows = jnp.minimum(base_rows, erows - wrow)
        pltpu.sync_copy(es_hbm.at[pl.ds(wrow, base_rows)], idxs)
        pltpu.sync_copy(ed_hbm.at[pl.ds(wrow, base_rows)], idxd)

        semg = (semg0, semg1)
        semw = (semw0, semw1)
        quads = ((as_hbm, gas_hbm, bas, idxs), (ad_hbm, gad_hbm, bad, idxd),
                 (bs_hbm, gbs_hbm, bbs, idxs), (bd_hbm, gbd_hbm, bbd, idxd))

        def gather(j, b):
            for tab, _, buf, idx in quads:
                pltpu.async_copy(tab.at[idx.at[j]], buf.at[b], semg[b])

        def wait_g(b):
            for tab, _, buf, idx in quads:
                pltpu.make_async_copy(tab.at[idx.at[0]], buf.at[b],
                                      semg[b]).wait()

        def writeback(j, b):
            ebase = (wrow + j) * GCH
            for _, out, buf, _i in quads:
                pltpu.async_copy(buf.at[b], out.at[pl.ds(ebase, GCH)], semw[b])

        def wait_w(b):
            ebase = wrow * GCH
            for _, out, buf, _i in quads:
                pltpu.make_async_copy(buf.at[b], out.at[pl.ds(ebase, GCH)],
                                      semw[b]).wait()

        gather(0, 0)
        gather(1, 1)

        def body(jp, _):
            wait_g(0)
            writeback(jp, 0)
            wait_g(1)
            writeback(jp + 1, 1)
            wait_w(0)

            @pl.when(jp + 2 < nrows)
            def _():
                gather(jp + 2, 0)
            wait_w(1)

            @pl.when(jp + 3 < nrows)
            def _():
                gather(jp + 3, 1)
            return 0

        lax.fori_loop(0, nrows // 2, lambda i, c: body(i * 2, c), 0, unroll=False)

    return gk


SCH = 128


def _sc_scatter_make(e, n_pad):
    erows = e // SCH
    base_rows = ((erows + SC_NS - 1) // SC_NS + 7) // 8 * 8
    half = erows // 2 // 8 * 8
    half_bp = ((max(half, erows - half) + SC_NS - 1) // SC_NS + 7) // 8 * 8
    out_rows = n_pad // SC_NS
    mesh = plsc.VectorSubcoreMesh(core_axis_name="c", subcore_axis_name="s")

    @functools.partial(
        pl.kernel,
        out_type=[jax.ShapeDtypeStruct((n_pad, SCW), jnp.float32)] * (NCHUNK + 1),
        mesh=mesh,
        scratch_types=[
            pltpu.VMEM((base_rows, SCH), jnp.int32),
            pltpu.VMEM((half_bp, SCH), jnp.int32),
            pltpu.VMEM((2, SCH, SCW), jnp.float32),
            pltpu.VMEM_SHARED((n_pad, SCW), jnp.float32),
            pltpu.SemaphoreType.DMA,
            pltpu.SemaphoreType.DMA,
            pltpu.SemaphoreType.DMA,
            pltpu.SemaphoreType.DMA,
        ],
    )
    def sk(v0, v1, v2, v3, v4, ed_hbm, z_hbm, o0, o1, o2a, o2b, o3, o4,
           idxf, idxh, rows, acc, r0, r1, s0, s1):
        c = lax.axis_index("c")
        s = lax.axis_index("s")
        pltpu.sync_copy(ed_hbm.at[pl.ds(s * base_rows, base_rows)], idxf)
        orow = s * out_rows
        semr = (r0, r1)
        sems = (s0, s1)

        def do_chunk(v_hbm, o_hbm, idx, tstart, nrows):
            pltpu.sync_copy(z_hbm.at[pl.ds(orow, out_rows)],
                            acc.at[pl.ds(orow, out_rows)])
            plsc.subcore_barrier()

            def load(j, b):
                pltpu.async_copy(v_hbm.at[pl.ds((tstart + j) * SCH, SCH)],
                                 rows.at[b], semr[b])

            def wait_r(b):
                pltpu.make_async_copy(v_hbm.at[pl.ds(0, SCH)], rows.at[b],
                                      semr[b]).wait()

            def scat(j, b):
                pltpu.async_copy(rows.at[b], acc.at[idx.at[j]], sems[b],
                                 add=True)

            def wait_s(b):
                pltpu.make_async_copy(rows.at[b], acc.at[idx.at[0]],
                                      sems[b]).wait()

            @pl.when(nrows > 0)
            def _():
                load(0, 0)
                load(1, 1)

            def body(jp, _):
                wait_r(0)
                scat(jp, 0)
                wait_r(1)
                scat(jp + 1, 1)
                wait_s(0)

                @pl.when(jp + 2 < nrows)
                def _():
                    load(jp + 2, 0)
                wait_s(1)

                @pl.when(jp + 3 < nrows)
                def _():
                    load(jp + 3, 1)
                return 0

            lax.fori_loop(0, nrows // 2, lambda i, cr: body(i * 2, cr), 0,
                          unroll=False)
            plsc.subcore_barrier()
            pltpu.sync_copy(acc.at[pl.ds(orow, out_rows)],
                            o_hbm.at[pl.ds(orow, out_rows)])
            plsc.subcore_barrier()

        def full(v_hbm, o_hbm):
            nrows = jnp.minimum(base_rows, erows - s * base_rows)
            do_chunk(v_hbm, o_hbm, idxf, s * base_rows, nrows)

        def partial(v_hbm, o_hbm, rlo, count):
            tstart = rlo + s * half_bp
            pltpu.sync_copy(ed_hbm.at[pl.ds(tstart, half_bp)], idxh)
            nrows = jnp.clip(count - s * half_bp, 0, half_bp)
            do_chunk(v_hbm, o_hbm, idxh, tstart, nrows)

        @pl.when(c == 0)
        def _():
            full(v0, o0)
            full(v1, o1)
            partial(v2, o2a, 0, half)

        @pl.when(c == 1)
        def _():
            full(v3, o3)
            full(v4, o4)
            partial(v2, o2b, half, erows - half)

    return sk


def _table_kernel(x_ref, wbd_ref, a_ref, b_ref):
    f32 = jnp.float32
    x = x_ref[...]
    y = jnp.dot(x, wbd_ref[...], preferred_element_type=f32)
    bl = jax.lax.bitcast_convert_type(y[:, 0:128], jnp.int32)
    bh = jax.lax.bitcast_convert_type(y[:, 128:256], jnp.int32)
    rl = bl + 0x7FFF + ((bl >> 16) & 1)
    rh = bh + 0x7FFF + ((bh >> 16) & 1)
    a_ref[...] = ((rl >> 16) & 0xFFFF) | (rh & jnp.int32(-65536))
    b_ref[:, 0:C] = x[:, 0:C]
    b_ref[:, C:SCW] = jnp.zeros((x.shape[0], SCW - C), f32)


def _edge_kernel(ed_ref, gas_ref, gad_ref, gbs_ref, gbd_ref, wr1_ref, br1_ref,
                 wr2_ref, br2_ref, w0_ref, b0_ref, lng_ref, lnb_ref, proj_ref,
                 ad_ref, g8_ref, w2bd_ref, expbig_ref, out0_ref, out1_ref,
                 out2_ref, out3_ref, out4_ref):
    f32 = jnp.float32
    dot = functools.partial(jnp.dot, preferred_element_type=f32)

    ed = ed_ref[...]
    rad = dot(ed, wr1_ref[...]) + br1_ref[...]
    rad = rad * jax.nn.sigmoid(rad)
    gate = dot(rad, wr2_ref[...]) + br2_ref[...]

    gs0 = gbs_ref[:, 0:C]
    gd0 = gbd_ref[:, 0:C]
    extra = dot(gs0, w0_ref[0:C, :]) + dot(gd0, w0_ref[C:2 * C, :]) + b0_ref[...]
    gating = extra[:, HEADS * A:]
    sg = jax.nn.sigmoid(gating)
    scal = gating * sg

    xa = extra[:, :HEADS * A]
    proj = proj_ref[...]
    mu = dot(xa, proj)
    cent = xa - mu
    var = dot(cent * cent, proj)
    xan = cent * jax.lax.rsqrt(var + 1e-5) * lng_ref[...] + lnb_ref[...]
    xsl = xan * jax.nn.sigmoid(xan) * 0.8 + 0.2 * xan
    alpha = dot(xsl, ad_ref[...])
    s = jnp.exp(alpha)

    mod = dot(gate * sg, g8_ref[...])
    wa = gas_ref[...]
    wd = gad_ref[...]
    bc = functools.partial(jax.lax.bitcast_convert_type, new_dtype=f32)
    msk = jnp.int32(-65536)
    sum0 = (bc(wa << 16) + bc(wd << 16)) * mod[:, 0:128]
    sum1 = (bc(wa & msk) + bc(wd & msk)) * mod[:, 128:256]
    hid = jnp.concatenate([scal, sum0, sum1], axis=1)
    val = dot(hid.astype(jnp.bfloat16), w2bd_ref[...])
    attn = val * dot(s, expbig_ref[...])

    out0_ref[...] = attn[:, 0:128]
    out1_ref[...] = attn[:, 128:256]
    out2_ref[...] = attn[:, 256:384]
    out3_ref[...] = attn[:, 384:512]
    out4_ref[:, 0:64] = attn[:, 512:576]
    out4_ref[:, 64:72] = s
    out4_ref[:, 72:SCW] = jnp.zeros((ed.shape[0], SCW - 72), f32)


def _node_kernel(n0_ref, n1_ref, n2a_ref, n2b_ref, n3_ref, n4_ref, wp_ref,
                 bp_ref, exp8_ref, out_ref):
    f32 = jnp.float32
    dot = functools.partial(jnp.dot, preferred_element_type=f32)
    n2 = n2a_ref[...] + n2b_ref[...]
    ns = (n0_ref[...], n1_ref[...], n2, n3_ref[...], n4_ref[...])
    asum = ns[4][:, 64:72]
    inv = 1.0 / (asum + 1e-16)
    inv64 = dot(inv, exp8_ref[...])
    for i in range(L2):
        l = L_OF_PY[i]
        acc = ns[i // 2][:, (i % 2) * 64:(i % 2) * 64 + 64] * inv64
        o = dot(acc, wp_ref[l])
        if i == 0:
            o = o + bp_ref[...]
        out_ref[:, i * OUT:(i + 1) * OUT] = o


def _run_table_kernel(x2, wbd8):
    n = x2.shape[0]
    nb = 1000 if n % 1000 == 0 else n
    return pl.pallas_call(
        _table_kernel,
        grid=(n // nb,),
        in_specs=[
            pl.BlockSpec((nb, L2 * C), lambda i: (i, 0)),
            pl.BlockSpec((L2 * C, 256), lambda i: (0, 0)),
        ],
        out_specs=[
            pl.BlockSpec((nb, 128), lambda i: (i, 0)),
            pl.BlockSpec((nb, SCW), lambda i: (i, 0)),
        ],
        out_shape=[
            jax.ShapeDtypeStruct((n, 128), jnp.int32),
            jax.ShapeDtypeStruct((n, SCW), jnp.float32),
        ],
    )(x2, wbd8)


def _run_edge_kernel(ed, gas, gad, gbs, gbd, Wr1, br1, Wr2, br2, W0, b0,
                     lng, lnb, proj, ad, g8, w2bd, expbig):
    e = ed.shape[0]
    grid = (e // EB,)
    full = lambda shape: pl.BlockSpec(shape, lambda i: (0,) * len(shape))
    return pl.pallas_call(
        _edge_kernel,
        grid=grid,
        in_specs=[
            pl.BlockSpec((EB, EDGE), lambda i: (i, 0)),
            pl.BlockSpec((EB, 128), lambda i: (i, 0)),
            pl.BlockSpec((EB, 128), lambda i: (i, 0)),
            pl.BlockSpec((EB, SCW), lambda i: (i, 0)),
            pl.BlockSpec((EB, SCW), lambda i: (i, 0)),
            full((EDGE, 64)), full((1, 64)), full((64, H)), full((1, H)),
            full((2 * C, EXTRA)), full((1, EXTRA)),
            full((1, HEADS * A)), full((1, HEADS * A)),
            full((HEADS * A, HEADS * A)), full((HEADS * A, HEADS)),
            full((H, 256)), full((L2 * C, 576)), full((HEADS, 576)),
        ],
        out_specs=[pl.BlockSpec((EB, SCW), lambda i: (i, 0))] * NCHUNK,
        out_shape=[jax.ShapeDtypeStruct((e, SCW), jnp.float32)] * NCHUNK,
    )(ed, gas, gad, gbs, gbd, Wr1, br1, Wr2, br2, W0, b0, lng, lnb, proj, ad,
      g8, w2bd, expbig)


def _run_node_kernel(nodes, Wp, bp, exp8):
    n = nodes[0].shape[0]
    grid = (n // NB,)
    full = lambda shape: pl.BlockSpec(shape, lambda i: (0,) * len(shape))
    return pl.pallas_call(
        _node_kernel,
        grid=grid,
        in_specs=[pl.BlockSpec((NB, SCW), lambda i: (i, 0))] * (NCHUNK + 1) + [
            full((3, HEADS * V, OUT)), full((1, OUT)), full((HEADS, HEADS * V)),
        ],
        out_specs=pl.BlockSpec((NB, L2 * OUT), lambda i: (i, 0)),
        out_shape=jax.ShapeDtypeStruct((n, L2 * OUT), jnp.float32),
    )(*nodes, Wp, bp, exp8)


def kernel(x_src, x_dst, edge_distance, edge_src, edge_dst, Wr1, br1, Wr2, br2,
           W1, W0, b0, ln_g, ln_b, alpha_dot, W2, Wp, bp):
    n = x_src.shape[0]
    e = edge_src.shape[0]
    f32 = jnp.float32

    lng = jnp.tile(ln_g, HEADS).reshape(1, HEADS * A)
    lnb = jnp.tile(ln_b, HEADS).reshape(1, HEADS * A)
    eyeh = jnp.eye(HEADS, dtype=f32)
    proj = jnp.kron(eyeh, jnp.ones((A, A), f32)) / A
    ad = (eyeh[:, None, :] * alpha_dot[:, :, None]).reshape(HEADS * A, HEADS)
    exp8 = jnp.kron(eyeh, jnp.ones((1, V), f32))
    eyec = jnp.eye(C, dtype=f32)
    g8 = jnp.concatenate([eyec] * (L2 - 1), axis=1)
    w1s_bd8 = jax.scipy.linalg.block_diag(
        *[W1[l, 0:C, :] for l in L_OF_PY[1:]])
    w1s_bd8 = jnp.pad(w1s_bd8, ((C, 0), (0, 0)))
    w1d_bd8 = jax.scipy.linalg.block_diag(
        *[W1[l, C:2 * C, :] for l in L_OF_PY[1:]])
    w1d_bd8 = jnp.pad(w1d_bd8, ((C, 0), (0, 0)))
    w2_bd = jax.scipy.linalg.block_diag(
        *[W2[l] for l in L_OF_PY]).astype(jnp.bfloat16)
    expbig = jnp.tile(exp8, (1, L2))

    tsa, tsb = _run_table_kernel(x_src.reshape(n, L2 * C), w1s_bd8)
    tda, tdb = _run_table_kernel(x_dst.reshape(n, L2 * C), w1d_bd8)

    erows = e // GCH
    g_base_rows = ((erows + SC_NW - 1) // SC_NW + 7) // 8 * 8
    g_pad = g_base_rows * SC_NW - erows
    es2 = jnp.pad(edge_src.reshape(erows, GCH), ((0, g_pad), (0, 0)))
    ed2 = jnp.pad(edge_dst.reshape(erows, GCH), ((0, g_pad), (0, 0)))
    gas, gad, gbs, gbd = _sc_gather_make(n, e)(tsa, tda, tsb, tdb, es2, ed2)

    vals = _run_edge_kernel(
        edge_distance, gas, gad, gbs, gbd, Wr1, br1.reshape(1, -1), Wr2,
        br2.reshape(1, -1), W0, b0.reshape(1, -1), lng, lnb, proj, ad, g8,
        w2_bd, expbig)

    srows = e // SCH
    s_base_rows = ((srows + SC_NS - 1) // SC_NS + 7) // 8 * 8
    s_pad = s_base_rows * SC_NS - srows
    eds2 = jnp.pad(edge_dst.reshape(srows, SCH), ((0, s_pad), (0, 0)))
    z = jnp.zeros((N_PAD, SCW), f32)
    nodes = _sc_scatter_make(e, N_PAD)(*vals, eds2, z)

    out = _run_node_kernel(nodes, Wp, bp.reshape(1, -1), exp8)
    return out[:n].reshape(n, L2, OUT)


# speedup vs baseline: 19.0185x; 1.0513x over previous
import functools

import jax
import jax.numpy as jnp
import numpy as np
from jax import lax
from jax.experimental import pallas as pl
from jax.experimental.pallas import tpu as pltpu
from jax.experimental.pallas import tpu_sc as plsc

N = 10000
E = 160000
L2 = 9
C = 32
H = 32
HEADS = 8
A = 16
V = 8
OUT = 32
EDGE = 64
EXTRA = HEADS * A + H
L_OF_PY = (0, 1, 1, 1, 2, 2, 2, 2, 2)

EB = 1280
NB = 1024
SCW = 128
NCHUNK = 5
N_PAD = 10240

SC_NC = 2
SC_NS = 16
SC_NW = SC_NC * SC_NS

GCH = 64
DG = 384


def _sc_gather_make(n, e):
    erows = e // GCH
    base_rows = ((erows + SC_NW - 1) // SC_NW + 7) // 8 * 8
    assert base_rows % 2 == 0 and (erows - base_rows * (SC_NW - 1)) % 2 == 0
    mesh = plsc.VectorSubcoreMesh(core_axis_name="c", subcore_axis_name="s")

    @functools.partial(
        pl.kernel,
        out_type=[jax.ShapeDtypeStruct((e, 128), jnp.int32),
                  jax.ShapeDtypeStruct((e, 128), jnp.int32),
                  jax.ShapeDtypeStruct((e, SCW), jnp.float32),
                  jax.ShapeDtypeStruct((e, SCW), jnp.float32)],
        mesh=mesh,
        scratch_types=[
            pltpu.VMEM((base_rows, GCH), jnp.int32),
            pltpu.VMEM((base_rows, GCH), jnp.int32),
            pltpu.VMEM((2, GCH, 128), jnp.int32),
            pltpu.VMEM((2, GCH, 128), jnp.int32),
            pltpu.VMEM((2, GCH, SCW), jnp.float32),
            pltpu.VMEM((2, GCH, SCW), jnp.float32),
            pltpu.SemaphoreType.DMA,
            pltpu.SemaphoreType.DMA,
            pltpu.SemaphoreType.DMA,
            pltpu.SemaphoreType.DMA,
        ],
    )
    def gk(as_hbm, ad_hbm, bs_hbm, bd_hbm, es_hbm, ed_hbm,
           gas_hbm, gad_hbm, gbs_hbm, gbd_hbm,
           idxs, idxd, bas, bad, bbs, bbd, semg0, semg1, semw0, semw1):
        wid = lax.axis_index("s") * SC_NC + lax.axis_index("c")
        wrow = wid * base_rows
        nrows = jnp.minimum(base_rows, erows - wrow)
        pltpu.sync_copy(es_hbm.at[pl.ds(wrow, base_rows)], idxs)
        pltpu.sync_copy(ed_hbm.at[pl.ds(wrow, base_rows)], idxd)

        semg = (semg0, semg1)
        semw = (semw0, semw1)
        quads = ((as_hbm, gas_hbm, bas, idxs), (ad_hbm, gad_hbm, bad, idxd),
                 (bs_hbm, gbs_hbm, bbs, idxs), (bd_hbm, gbd_hbm, bbd, idxd))

        def gather(j, b):
            for tab, _, buf, idx in quads:
                pltpu.async_copy(tab.at[idx.at[j]], buf.at[b], semg[b])

        def wait_g(b):
            for tab, _, buf, idx in quads:
                pltpu.make_async_copy(tab.at[idx.at[0]], buf.at[b],
                                      semg[b]).wait()

        def writeback(j, b):
            ebase = (wrow + j) * GCH
            for _, out, buf, _i in quads:
                pltpu.async_copy(buf.at[b], out.at[pl.ds(ebase, GCH)], semw[b])

        def wait_w(b):
            ebase = wrow * GCH
            for _, out, buf, _i in quads:
                pltpu.make_async_copy(buf.at[b], out.at[pl.ds(ebase, GCH)],
                                      semw[b]).wait()

        gather(0, 0)
        gather(1, 1)

        def body(jp, _):
            wait_g(0)
            writeback(jp, 0)
            wait_g(1)
            writeback(jp + 1, 1)
            wait_w(0)

            @pl.when(jp + 2 < nrows)
            def _():
                gather(jp + 2, 0)
            wait_w(1)

            @pl.when(jp + 3 < nrows)
            def _():
                gather(jp + 3, 1)
            return 0

        lax.fori_loop(0, nrows // 2, lambda i, c: body(i * 2, c), 0, unroll=False)

    return gk


SCH = 128


def _sc_scatter_make(e, n_pad):
    erows = e // SCH
    base_rows = ((erows + SC_NS - 1) // SC_NS + 7) // 8 * 8
    half = erows // 2 // 8 * 8
    half_bp = ((max(half, erows - half) + SC_NS - 1) // SC_NS + 7) // 8 * 8
    out_rows = n_pad // SC_NS
    mesh = plsc.VectorSubcoreMesh(core_axis_name="c", subcore_axis_name="s")

    @functools.partial(
        pl.kernel,
        out_type=[jax.ShapeDtypeStruct((n_pad, SCW), jnp.float32)] * (NCHUNK + 1),
        mesh=mesh,
        scratch_types=[
            pltpu.VMEM((base_rows, SCH), jnp.int32),
            pltpu.VMEM((half_bp, SCH), jnp.int32),
            pltpu.VMEM((2, SCH, SCW), jnp.float32),
            pltpu.VMEM_SHARED((n_pad, SCW), jnp.float32),
            pltpu.SemaphoreType.DMA,
            pltpu.SemaphoreType.DMA,
            pltpu.SemaphoreType.DMA,
            pltpu.SemaphoreType.DMA,
        ],
    )
    def sk(v0, v1, v2, v3, v4, ed_hbm, z_hbm, o0, o1, o2a, o2b, o3, o4,
           idxf, idxh, rows, acc, r0, r1, s0, s1):
        c = lax.axis_index("c")
        s = lax.axis_index("s")
        pltpu.sync_copy(ed_hbm.at[pl.ds(s * base_rows, base_rows)], idxf)
        orow = s * out_rows
        semr = (r0, r1)
        sems = (s0, s1)

        def do_chunk(v_hbm, o_hbm, idx, tstart, nrows):
            pltpu.sync_copy(z_hbm.at[pl.ds(orow, out_rows)],
                            acc.at[pl.ds(orow, out_rows)])
            plsc.subcore_barrier()

            def load(j, b):
                pltpu.async_copy(v_hbm.at[pl.ds((tstart + j) * SCH, SCH)],
                                 rows.at[b], semr[b])

            def wait_r(b):
                pltpu.make_async_copy(v_hbm.at[pl.ds(0, SCH)], rows.at[b],
                                      semr[b]).wait()

            def scat(j, b):
                pltpu.async_copy(rows.at[b], acc.at[idx.at[j]], sems[b],
                                 add=True)

            def wait_s(b):
                pltpu.make_async_copy(rows.at[b], acc.at[idx.at[0]],
                                      sems[b]).wait()

            @pl.when(nrows > 0)
            def _():
                load(0, 0)
                load(1, 1)

            def body(jp, _):
                wait_r(0)
                scat(jp, 0)
                wait_r(1)
                scat(jp + 1, 1)
                wait_s(0)

                @pl.when(jp + 2 < nrows)
                def _():
                    load(jp + 2, 0)
                wait_s(1)

                @pl.when(jp + 3 < nrows)
                def _():
                    load(jp + 3, 1)
                return 0

            lax.fori_loop(0, nrows // 2, lambda i, cr: body(i * 2, cr), 0,
                          unroll=False)
            plsc.subcore_barrier()
            pltpu.sync_copy(acc.at[pl.ds(orow, out_rows)],
                            o_hbm.at[pl.ds(orow, out_rows)])
            plsc.subcore_barrier()

        def full(v_hbm, o_hbm):
            nrows = jnp.minimum(base_rows, erows - s * base_rows)
            do_chunk(v_hbm, o_hbm, idxf, s * base_rows, nrows)

        def partial(v_hbm, o_hbm, rlo, count):
            tstart = rlo + s * half_bp
            pltpu.sync_copy(ed_hbm.at[pl.ds(tstart, half_bp)], idxh)
            nrows = jnp.clip(count - s * half_bp, 0, half_bp)
            do_chunk(v_hbm, o_hbm, idxh, tstart, nrows)

        @pl.when(c == 0)
        def _():
            full(v0, o0)
            full(v1, o1)
            partial(v2, o2a, 0, half)

        @pl.when(c == 1)
        def _():
            full(v3, o3)
            full(v4, o4)
            partial(v2, o2b, half, erows - half)

    return sk


def _table_kernel(xt_ref, wbd_ref, sel_ref, a_ref, b_ref):
    f32 = jnp.float32
    xt = xt_ref[...]
    dims = (((0,), (0,)), ((), ()))
    y = jax.lax.dot_general(xt, wbd_ref[...], dims,
                            preferred_element_type=f32)
    x0 = jax.lax.dot_general(xt, sel_ref[...], dims,
                             preferred_element_type=f32)
    bl = jax.lax.bitcast_convert_type(y[:, 0:128], jnp.int32)
    bh = jax.lax.bitcast_convert_type(y[:, 128:256], jnp.int32)
    rl = bl + 0x7FFF + ((bl >> 16) & 1)
    rh = bh + 0x7FFF + ((bh >> 16) & 1)
    a_ref[...] = ((rl >> 16) & 0xFFFF) | (rh & jnp.int32(-65536))
    b_ref[:, 0:C] = x0
    b_ref[:, C:SCW] = jnp.zeros((x0.shape[0], SCW - C), f32)


def _edge_kernel(edt_ref, gas_ref, gad_ref, gbs_ref, gbd_ref, wr1_ref, br1_ref,
                 wr2_ref, br2_ref, w0_ref, b0_ref, lng_ref, lnb_ref, proj_ref,
                 ad_ref, g8_ref, w2bd_ref, expbig_ref, out0_ref, out1_ref,
                 out2_ref, out3_ref, out4_ref):
    f32 = jnp.float32
    dot = functools.partial(jnp.dot, preferred_element_type=f32)

    rad = jax.lax.dot_general(edt_ref[...], wr1_ref[...], (((0,), (0,)), ((), ())),
                              preferred_element_type=f32) + br1_ref[...]
    rad = rad * jax.nn.sigmoid(rad)
    gate = dot(rad, wr2_ref[...]) + br2_ref[...]

    gs0 = gbs_ref[:, 0:C]
    gd0 = gbd_ref[:, 0:C]
    extra = dot(gs0, w0_ref[0:C, :]) + dot(gd0, w0_ref[C:2 * C, :]) + b0_ref[...]
    gating = extra[:, HEADS * A:]
    sg = jax.nn.sigmoid(gating)
    scal = gating * sg

    xa = extra[:, :HEADS * A]
    proj = proj_ref[...]
    mu = dot(xa, proj)
    cent = xa - mu
    var = dot(cent * cent, proj)
    xan = cent * jax.lax.rsqrt(var + 1e-5) * lng_ref[...] + lnb_ref[...]
    xsl = xan * jax.nn.sigmoid(xan) * 0.8 + 0.2 * xan
    alpha = dot(xsl, ad_ref[...])
    s = jnp.exp(alpha)

    mod = dot(gate * sg, g8_ref[...])
    wa = gas_ref[...]
    wd = gad_ref[...]
    bc = functools.partial(jax.lax.bitcast_convert_type, new_dtype=f32)
    msk = jnp.int32(-65536)
    sum0 = (bc(wa << 16) + bc(wd << 16)) * mod[:, 0:128]
    sum1 = (bc(wa & msk) + bc(wd & msk)) * mod[:, 128:256]
    hid = jnp.concatenate([scal, sum0, sum1], axis=1)
    val = dot(hid.astype(jnp.bfloat16), w2bd_ref[...])
    attn = val * dot(s, expbig_ref[...])

    out0_ref[...] = attn[:, 0:128]
    out1_ref[...] = attn[:, 128:256]
    out2_ref[...] = attn[:, 256:384]
    out3_ref[...] = attn[:, 384:512]
    out4_ref[:, 0:64] = attn[:, 512:576]
    out4_ref[:, 64:72] = s
    out4_ref[:, 72:SCW] = jnp.zeros((gate.shape[0], SCW - 72), f32)


def _node_kernel(n0_ref, n1_ref, n2a_ref, n2b_ref, n3_ref, n4_ref, wp_ref,
                 bp_ref, exp8_ref, out_ref):
    f32 = jnp.float32
    dot = functools.partial(jnp.dot, preferred_element_type=f32)
    n2 = n2a_ref[...] + n2b_ref[...]
    ns = (n0_ref[...], n1_ref[...], n2, n3_ref[...], n4_ref[...])
    asum = ns[4][:, 64:72]
    inv = 1.0 / (asum + 1e-16)
    inv64 = dot(inv, exp8_ref[...])
    for i in range(L2):
        l = L_OF_PY[i]
        acc = ns[i // 2][:, (i % 2) * 64:(i % 2) * 64 + 64] * inv64
        o = dot(acc, wp_ref[l])
        if i == 0:
            o = o + bp_ref[...]
        out_ref[:, i * OUT:(i + 1) * OUT] = o


def _run_table_kernel(xt, wbd8, sel):
    n = xt.shape[1]
    nb = n
    return pl.pallas_call(
        _table_kernel,
        grid=(n // nb,),
        in_specs=[
            pl.BlockSpec((L2 * C, nb), lambda i: (0, i)),
            pl.BlockSpec((L2 * C, 256), lambda i: (0, 0)),
            pl.BlockSpec((L2 * C, C), lambda i: (0, 0)),
        ],
        out_specs=[
            pl.BlockSpec((nb, 128), lambda i: (i, 0)),
            pl.BlockSpec((nb, SCW), lambda i: (i, 0)),
        ],
        out_shape=[
            jax.ShapeDtypeStruct((n, 128), jnp.int32),
            jax.ShapeDtypeStruct((n, SCW), jnp.float32),
        ],
    )(xt, wbd8, sel)


def _run_edge_kernel(edt, gas, gad, gbs, gbd, Wr1, br1, Wr2, br2, W0, b0,
                     lng, lnb, proj, ad, g8, w2bd, expbig):
    e = edt.shape[1]
    grid = (e // EB,)
    full = lambda shape: pl.BlockSpec(shape, lambda i: (0,) * len(shape))
    return pl.pallas_call(
        _edge_kernel,
        grid=grid,
        in_specs=[
            pl.BlockSpec((EDGE, EB), lambda i: (0, i)),
            pl.BlockSpec((EB, 128), lambda i: (i, 0)),
            pl.BlockSpec((EB, 128), lambda i: (i, 0)),
            pl.BlockSpec((EB, SCW), lambda i: (i, 0)),
            pl.BlockSpec((EB, SCW), lambda i: (i, 0)),
            full((EDGE, 64)), full((1, 64)), full((64, H)), full((1, H)),
            full((2 * C, EXTRA)), full((1, EXTRA)),
            full((1, HEADS * A)), full((1, HEADS * A)),
            full((HEADS * A, HEADS * A)), full((HEADS * A, HEADS)),
            full((H, 256)), full((L2 * C, 576)), full((HEADS, 576)),
        ],
        out_specs=[pl.BlockSpec((EB, SCW), lambda i: (i, 0))] * NCHUNK,
        out_shape=[jax.ShapeDtypeStruct((e, SCW), jnp.float32)] * NCHUNK,
    )(edt, gas, gad, gbs, gbd, Wr1, br1, Wr2, br2, W0, b0, lng, lnb, proj, ad,
      g8, w2bd, expbig)


def _run_node_kernel(nodes, Wp, bp, exp8):
    n = nodes[0].shape[0]
    grid = (n // NB,)
    full = lambda shape: pl.BlockSpec(shape, lambda i: (0,) * len(shape))
    return pl.pallas_call(
        _node_kernel,
        grid=grid,
        in_specs=[pl.BlockSpec((NB, SCW), lambda i: (i, 0))] * (NCHUNK + 1) + [
            full((3, HEADS * V, OUT)), full((1, OUT)), full((HEADS, HEADS * V)),
        ],
        out_specs=pl.BlockSpec((NB, L2 * OUT), lambda i: (i, 0)),
        out_shape=jax.ShapeDtypeStruct((n, L2 * OUT), jnp.float32),
    )(*nodes, Wp, bp, exp8)


def kernel(x_src, x_dst, edge_distance, edge_src, edge_dst, Wr1, br1, Wr2, br2,
           W1, W0, b0, ln_g, ln_b, alpha_dot, W2, Wp, bp):
    n = x_src.shape[0]
    e = edge_src.shape[0]
    f32 = jnp.float32

    lng = jnp.tile(ln_g, HEADS).reshape(1, HEADS * A)
    lnb = jnp.tile(ln_b, HEADS).reshape(1, HEADS * A)
    eyeh = jnp.eye(HEADS, dtype=f32)
    proj = jnp.kron(eyeh, jnp.ones((A, A), f32)) / A
    ad = (eyeh[:, None, :] * alpha_dot[:, :, None]).reshape(HEADS * A, HEADS)
    exp8 = jnp.kron(eyeh, jnp.ones((1, V), f32))
    eyec = jnp.eye(C, dtype=f32)
    g8 = jnp.concatenate([eyec] * (L2 - 1), axis=1)
    w1s_bd8 = jax.scipy.linalg.block_diag(
        *[W1[l, 0:C, :] for l in L_OF_PY[1:]])
    w1s_bd8 = jnp.pad(w1s_bd8, ((C, 0), (0, 0)))
    w1d_bd8 = jax.scipy.linalg.block_diag(
        *[W1[l, C:2 * C, :] for l in L_OF_PY[1:]])
    w1d_bd8 = jnp.pad(w1d_bd8, ((C, 0), (0, 0)))
    w2_bd = jax.scipy.linalg.block_diag(
        *[W2[l] for l in L_OF_PY]).astype(jnp.bfloat16)
    expbig = jnp.tile(exp8, (1, L2))

    sel = jnp.pad(eyec, ((0, (L2 - 1) * C), (0, 0)))

    xst = x_src.transpose(1, 2, 0).reshape(L2 * C, n)
    xdt = x_dst.transpose(1, 2, 0).reshape(L2 * C, n)
    tsa, tsb = _run_table_kernel(xst, w1s_bd8, sel)
    tda, tdb = _run_table_kernel(xdt, w1d_bd8, sel)

    erows = e // GCH
    g_base_rows = ((erows + SC_NW - 1) // SC_NW + 7) // 8 * 8
    g_pad = g_base_rows * SC_NW - erows
    es2 = jnp.pad(edge_src.reshape(erows, GCH), ((0, g_pad), (0, 0)))
    ed2 = jnp.pad(edge_dst.reshape(erows, GCH), ((0, g_pad), (0, 0)))
    gas, gad, gbs, gbd = _sc_gather_make(n, e)(tsa, tda, tsb, tdb, es2, ed2)

    vals = _run_edge_kernel(
        edge_distance.T, gas, gad, gbs, gbd, Wr1, br1.reshape(1, -1), Wr2,
        br2.reshape(1, -1), W0, b0.reshape(1, -1), lng, lnb, proj, ad, g8,
        w2_bd, expbig)

    srows = e // SCH
    s_base_rows = ((srows + SC_NS - 1) // SC_NS + 7) // 8 * 8
    s_pad = s_base_rows * SC_NS - srows
    eds2 = jnp.pad(edge_dst.reshape(srows, SCH), ((0, s_pad), (0, 0)))
    z = jnp.zeros((N_PAD, SCW), f32)
    nodes = _sc_scatter_make(e, N_PAD)(*vals, eds2, z)

    out = _run_node_kernel(nodes, Wp, bp.reshape(1, -1), exp8)
    return out[:n].reshape(n, L2, OUT)


# speedup vs baseline: 22.5456x; 1.1855x over previous
import functools

import jax
import jax.numpy as jnp
import numpy as np
from jax import lax
from jax.experimental import pallas as pl
from jax.experimental.pallas import tpu as pltpu
from jax.experimental.pallas import tpu_sc as plsc

N = 10000
E = 160000
L2 = 9
C = 32
H = 32
HEADS = 8
A = 16
V = 8
OUT = 32
EDGE = 64
EXTRA = HEADS * A + H
L_OF_PY = (0, 1, 1, 1, 2, 2, 2, 2, 2)

EB = 1280
NB = 1024
SCW = 128
NCHUNK = 5
N_PAD = 10240

SC_NC = 2
SC_NS = 16
SC_NW = SC_NC * SC_NS

GCH = 64
DG = 384


def _sc_gather_make(n, e):
    erows = e // GCH
    base_rows = ((erows + SC_NW - 1) // SC_NW + 7) // 8 * 8
    assert base_rows % 2 == 0 and (erows - base_rows * (SC_NW - 1)) % 2 == 0
    mesh = plsc.VectorSubcoreMesh(core_axis_name="c", subcore_axis_name="s")

    @functools.partial(
        pl.kernel,
        out_type=[jax.ShapeDtypeStruct((e, 128), jnp.int32),
                  jax.ShapeDtypeStruct((e, 128), jnp.int32),
                  jax.ShapeDtypeStruct((e, SCW), jnp.float32),
                  jax.ShapeDtypeStruct((e, SCW), jnp.float32)],
        mesh=mesh,
        scratch_types=[
            pltpu.VMEM((base_rows, GCH), jnp.int32),
            pltpu.VMEM((base_rows, GCH), jnp.int32),
            pltpu.VMEM((2, GCH, 128), jnp.int32),
            pltpu.VMEM((2, GCH, 128), jnp.int32),
            pltpu.VMEM((2, GCH, SCW), jnp.float32),
            pltpu.VMEM((2, GCH, SCW), jnp.float32),
            pltpu.SemaphoreType.DMA,
            pltpu.SemaphoreType.DMA,
            pltpu.SemaphoreType.DMA,
            pltpu.SemaphoreType.DMA,
        ],
    )
    def gk(as_hbm, ad_hbm, bs_hbm, bd_hbm, es_hbm, ed_hbm,
           gas_hbm, gad_hbm, gbs_hbm, gbd_hbm,
           idxs, idxd, bas, bad, bbs, bbd, semg0, semg1, semw0, semw1):
        wid = lax.axis_index("s") * SC_NC + lax.axis_index("c")
        wrow = wid * base_rows
        nrows = jnp.clip(erows - wrow, 0, base_rows)
        pltpu.sync_copy(es_hbm.at[pl.ds(wrow, base_rows)], idxs)
        pltpu.sync_copy(ed_hbm.at[pl.ds(wrow, base_rows)], idxd)

        semg = (semg0, semg1)
        semw = (semw0, semw1)
        quads = ((as_hbm, gas_hbm, bas, idxs), (ad_hbm, gad_hbm, bad, idxd),
                 (bs_hbm, gbs_hbm, bbs, idxs), (bd_hbm, gbd_hbm, bbd, idxd))

        def gather(j, b):
            for tab, _, buf, idx in quads:
                pltpu.async_copy(tab.at[idx.at[j]], buf.at[b], semg[b])

        def wait_g(b):
            for tab, _, buf, idx in quads:
                pltpu.make_async_copy(tab.at[idx.at[0]], buf.at[b],
                                      semg[b]).wait()

        def writeback(j, b):
            ebase = (wrow + j) * GCH
            for _, out, buf, _i in quads:
                pltpu.async_copy(buf.at[b], out.at[pl.ds(ebase, GCH)], semw[b])

        def wait_w(b):
            ebase = wrow * GCH
            for _, out, buf, _i in quads:
                pltpu.make_async_copy(buf.at[b], out.at[pl.ds(ebase, GCH)],
                                      semw[b]).wait()

        @pl.when(nrows > 0)
        def _():
            gather(0, 0)
            gather(1, 1)

        def body(jp, _):
            wait_g(0)
            writeback(jp, 0)
            wait_g(1)
            writeback(jp + 1, 1)
            wait_w(0)

            @pl.when(jp + 2 < nrows)
            def _():
                gather(jp + 2, 0)
            wait_w(1)

            @pl.when(jp + 3 < nrows)
            def _():
                gather(jp + 3, 1)
            return 0

        lax.fori_loop(0, nrows // 2, lambda i, c: body(i * 2, c), 0, unroll=False)

    return gk


SCH = 128


def _sc_scatter_make(e, n_pad):
    erows = e // SCH
    base_rows = ((erows + SC_NS - 1) // SC_NS + 7) // 8 * 8
    half = erows // 2 // 8 * 8
    half_bp = ((max(half, erows - half) + SC_NS - 1) // SC_NS + 7) // 8 * 8
    out_rows = n_pad // SC_NS
    mesh = plsc.VectorSubcoreMesh(core_axis_name="c", subcore_axis_name="s")

    @functools.partial(
        pl.kernel,
        out_type=[jax.ShapeDtypeStruct((n_pad, SCW), jnp.float32)] * (NCHUNK + 1),
        mesh=mesh,
        scratch_types=[
            pltpu.VMEM((base_rows, SCH), jnp.int32),
            pltpu.VMEM((half_bp, SCH), jnp.int32),
            pltpu.VMEM((2, SCH, SCW), jnp.float32),
            pltpu.VMEM_SHARED((n_pad, SCW), jnp.float32),
            pltpu.SemaphoreType.DMA,
            pltpu.SemaphoreType.DMA,
            pltpu.SemaphoreType.DMA,
            pltpu.SemaphoreType.DMA,
        ],
    )
    def sk(v0, v1, v2, v3, v4, ed_hbm, z_hbm, o0, o1, o2a, o2b, o3, o4,
           idxf, idxh, rows, acc, r0, r1, s0, s1):
        c = lax.axis_index("c")
        s = lax.axis_index("s")
        pltpu.sync_copy(ed_hbm.at[pl.ds(s * base_rows, base_rows)], idxf)
        orow = s * out_rows
        semr = (r0, r1)
        sems = (s0, s1)

        def do_chunk(v_hbm, o_hbm, idx, tstart, nrows):
            pltpu.sync_copy(z_hbm.at[pl.ds(orow, out_rows)],
                            acc.at[pl.ds(orow, out_rows)])
            plsc.subcore_barrier()

            def load(j, b):
                pltpu.async_copy(v_hbm.at[pl.ds((tstart + j) * SCH, SCH)],
                                 rows.at[b], semr[b])

            def wait_r(b):
                pltpu.make_async_copy(v_hbm.at[pl.ds(0, SCH)], rows.at[b],
                                      semr[b]).wait()

            def scat(j, b):
                pltpu.async_copy(rows.at[b], acc.at[idx.at[j]], sems[b],
                                 add=True)

            def wait_s(b):
                pltpu.make_async_copy(rows.at[b], acc.at[idx.at[0]],
                                      sems[b]).wait()

            @pl.when(nrows > 0)
            def _():
                load(0, 0)
                load(1, 1)

            def body(jp, _):
                wait_r(0)
                scat(jp, 0)
                wait_r(1)
                scat(jp + 1, 1)
                wait_s(0)

                @pl.when(jp + 2 < nrows)
                def _():
                    load(jp + 2, 0)
                wait_s(1)

                @pl.when(jp + 3 < nrows)
                def _():
                    load(jp + 3, 1)
                return 0

            lax.fori_loop(0, nrows // 2, lambda i, cr: body(i * 2, cr), 0,
                          unroll=False)
            plsc.subcore_barrier()
            pltpu.sync_copy(acc.at[pl.ds(orow, out_rows)],
                            o_hbm.at[pl.ds(orow, out_rows)])
            plsc.subcore_barrier()

        def full(v_hbm, o_hbm):
            nrows = jnp.minimum(base_rows, erows - s * base_rows)
            do_chunk(v_hbm, o_hbm, idxf, s * base_rows, nrows)

        def partial(v_hbm, o_hbm, rlo, count):
            tstart = rlo + s * half_bp
            nrows = jnp.clip(count - s * half_bp, 0, half_bp)

            @pl.when(nrows > 0)
            def _():
                pltpu.sync_copy(ed_hbm.at[pl.ds(tstart, half_bp)], idxh)
            do_chunk(v_hbm, o_hbm, idxh, tstart, nrows)

        @pl.when(c == 0)
        def _():
            full(v0, o0)
            full(v1, o1)
            partial(v2, o2a, 0, half)

        @pl.when(c == 1)
        def _():
            full(v3, o3)
            full(v4, o4)
            partial(v2, o2b, half, erows - half)

    return sk


def _table_kernel(xt_ref, wbd_ref, sel_ref, a_ref, b_ref):
    f32 = jnp.float32
    xt = xt_ref[...]
    dims = (((0,), (0,)), ((), ()))
    y = jax.lax.dot_general(xt, wbd_ref[...], dims,
                            preferred_element_type=f32)
    x0 = jax.lax.dot_general(xt, sel_ref[...], dims,
                             preferred_element_type=f32)
    bl = jax.lax.bitcast_convert_type(y[:, 0:128], jnp.int32)
    bh = jax.lax.bitcast_convert_type(y[:, 128:256], jnp.int32)
    rl = bl + 0x7FFF + ((bl >> 16) & 1)
    rh = bh + 0x7FFF + ((bh >> 16) & 1)
    a_ref[...] = ((rl >> 16) & 0xFFFF) | (rh & jnp.int32(-65536))
    b_ref[:, 0:C] = x0
    b_ref[:, C:SCW] = jnp.zeros((x0.shape[0], SCW - C), f32)


def _edge_kernel(edt_ref, gas_ref, gad_ref, gbs_ref, gbd_ref, wr1_ref, br1_ref,
                 wr2_ref, br2_ref, w0_ref, b0_ref, lng_ref, lnb_ref, proj_ref,
                 ad_ref, g8_ref, w2bd_ref, expbig_ref, out0_ref, out1_ref,
                 out2_ref, out3_ref, out4_ref):
    f32 = jnp.float32
    dot = functools.partial(jnp.dot, preferred_element_type=f32)

    rad = jax.lax.dot_general(edt_ref[...], wr1_ref[...], (((0,), (0,)), ((), ())),
                              preferred_element_type=f32) + br1_ref[...]
    rad = rad * jax.nn.sigmoid(rad)
    gate = dot(rad, wr2_ref[...]) + br2_ref[...]

    gs0 = gbs_ref[:, 0:C]
    gd0 = gbd_ref[:, 0:C]
    extra = dot(gs0, w0_ref[0:C, :]) + dot(gd0, w0_ref[C:2 * C, :]) + b0_ref[...]
    gating = extra[:, HEADS * A:]
    sg = jax.nn.sigmoid(gating)
    scal = gating * sg

    xa = extra[:, :HEADS * A]
    proj = proj_ref[...]
    mu = dot(xa, proj)
    cent = xa - mu
    var = dot(cent * cent, proj)
    xan = cent * jax.lax.rsqrt(var + 1e-5) * lng_ref[...] + lnb_ref[...]
    xsl = xan * jax.nn.sigmoid(xan) * 0.8 + 0.2 * xan
    alpha = dot(xsl, ad_ref[...])
    s = jnp.exp(alpha)

    mod = dot(gate * sg, g8_ref[...])
    wa = gas_ref[...]
    wd = gad_ref[...]
    bc = functools.partial(jax.lax.bitcast_convert_type, new_dtype=f32)
    msk = jnp.int32(-65536)
    sum0 = (bc(wa << 16) + bc(wd << 16)) * mod[:, 0:128]
    sum1 = (bc(wa & msk) + bc(wd & msk)) * mod[:, 128:256]
    hid = jnp.concatenate([scal, sum0, sum1], axis=1)
    val = dot(hid.astype(jnp.bfloat16), w2bd_ref[...])
    attn = val * dot(s, expbig_ref[...])

    out0_ref[...] = attn[:, 0:128]
    out1_ref[...] = attn[:, 128:256]
    out2_ref[...] = attn[:, 256:384]
    out3_ref[...] = attn[:, 384:512]
    out4_ref[:, 0:64] = attn[:, 512:576]
    out4_ref[:, 64:72] = s
    out4_ref[:, 72:SCW] = jnp.zeros((gate.shape[0], SCW - 72), f32)


def _node_kernel(*refs):
    (h0_0, h0_1, h0_2a, h0_2b, h0_3, h0_4,
     h1_0, h1_1, h1_2a, h1_2b, h1_3, h1_4,
     wp_ref, bp_ref, exp8_ref, out_ref) = refs
    f32 = jnp.float32
    dot = functools.partial(jnp.dot, preferred_element_type=f32)
    n2 = h0_2a[...] + h0_2b[...] + h1_2a[...] + h1_2b[...]
    ns = (h0_0[...] + h1_0[...], h0_1[...] + h1_1[...], n2,
          h0_3[...] + h1_3[...], h0_4[...] + h1_4[...])
    asum = ns[4][:, 64:72]
    inv = 1.0 / (asum + 1e-16)
    inv64 = dot(inv, exp8_ref[...])
    for i in range(L2):
        l = L_OF_PY[i]
        acc = ns[i // 2][:, (i % 2) * 64:(i % 2) * 64 + 64] * inv64
        o = dot(acc, wp_ref[l])
        if i == 0:
            o = o + bp_ref[...]
        out_ref[:, i * OUT:(i + 1) * OUT] = o


def _run_table_kernel(xt, wbd8, sel):
    n = xt.shape[1]
    nb = n
    return pl.pallas_call(
        _table_kernel,
        grid=(n // nb,),
        in_specs=[
            pl.BlockSpec((L2 * C, nb), lambda i: (0, i)),
            pl.BlockSpec((L2 * C, 256), lambda i: (0, 0)),
            pl.BlockSpec((L2 * C, C), lambda i: (0, 0)),
        ],
        out_specs=[
            pl.BlockSpec((nb, 128), lambda i: (i, 0)),
            pl.BlockSpec((nb, SCW), lambda i: (i, 0)),
        ],
        out_shape=[
            jax.ShapeDtypeStruct((n, 128), jnp.int32),
            jax.ShapeDtypeStruct((n, SCW), jnp.float32),
        ],
    )(xt, wbd8, sel)


def _run_edge_kernel(edt, gas, gad, gbs, gbd, Wr1, br1, Wr2, br2, W0, b0,
                     lng, lnb, proj, ad, g8, w2bd, expbig, off=0):
    e = gas.shape[0]
    grid = (e // EB,)
    full = lambda shape: pl.BlockSpec(shape, lambda i: (0,) * len(shape))
    return pl.pallas_call(
        _edge_kernel,
        grid=grid,
        in_specs=[
            pl.BlockSpec((EDGE, EB), lambda i: (0, i + off)),
            pl.BlockSpec((EB, 128), lambda i: (i, 0)),
            pl.BlockSpec((EB, 128), lambda i: (i, 0)),
            pl.BlockSpec((EB, SCW), lambda i: (i, 0)),
            pl.BlockSpec((EB, SCW), lambda i: (i, 0)),
            full((EDGE, 64)), full((1, 64)), full((64, H)), full((1, H)),
            full((2 * C, EXTRA)), full((1, EXTRA)),
            full((1, HEADS * A)), full((1, HEADS * A)),
            full((HEADS * A, HEADS * A)), full((HEADS * A, HEADS)),
            full((H, 256)), full((L2 * C, 576)), full((HEADS, 576)),
        ],
        out_specs=[pl.BlockSpec((EB, SCW), lambda i: (i, 0))] * NCHUNK,
        out_shape=[jax.ShapeDtypeStruct((e, SCW), jnp.float32)] * NCHUNK,
    )(edt, gas, gad, gbs, gbd, Wr1, br1, Wr2, br2, W0, b0, lng, lnb, proj, ad,
      g8, w2bd, expbig)


def _run_node_kernel(nodes, Wp, bp, exp8):
    n = nodes[0].shape[0]
    grid = (n // NB,)
    full = lambda shape: pl.BlockSpec(shape, lambda i: (0,) * len(shape))
    return pl.pallas_call(
        _node_kernel,
        grid=grid,
        in_specs=[pl.BlockSpec((NB, SCW), lambda i: (i, 0))] * (2 * (NCHUNK + 1)) + [
            full((3, HEADS * V, OUT)), full((1, OUT)), full((HEADS, HEADS * V)),
        ],
        out_specs=pl.BlockSpec((NB, L2 * OUT), lambda i: (i, 0)),
        out_shape=jax.ShapeDtypeStruct((n, L2 * OUT), jnp.float32),
    )(*nodes, Wp, bp, exp8)


def kernel(x_src, x_dst, edge_distance, edge_src, edge_dst, Wr1, br1, Wr2, br2,
           W1, W0, b0, ln_g, ln_b, alpha_dot, W2, Wp, bp):
    n = x_src.shape[0]
    e = edge_src.shape[0]
    f32 = jnp.float32

    lng = jnp.tile(ln_g, HEADS).reshape(1, HEADS * A)
    lnb = jnp.tile(ln_b, HEADS).reshape(1, HEADS * A)
    eyeh = jnp.eye(HEADS, dtype=f32)
    proj = jnp.kron(eyeh, jnp.ones((A, A), f32)) / A
    ad = (eyeh[:, None, :] * alpha_dot[:, :, None]).reshape(HEADS * A, HEADS)
    exp8 = jnp.kron(eyeh, jnp.ones((1, V), f32))
    eyec = jnp.eye(C, dtype=f32)
    g8 = jnp.concatenate([eyec] * (L2 - 1), axis=1)
    w1s_bd8 = jax.scipy.linalg.block_diag(
        *[W1[l, 0:C, :] for l in L_OF_PY[1:]])
    w1s_bd8 = jnp.pad(w1s_bd8, ((C, 0), (0, 0)))
    w1d_bd8 = jax.scipy.linalg.block_diag(
        *[W1[l, C:2 * C, :] for l in L_OF_PY[1:]])
    w1d_bd8 = jnp.pad(w1d_bd8, ((C, 0), (0, 0)))
    w2_bd = jax.scipy.linalg.block_diag(
        *[W2[l] for l in L_OF_PY]).astype(jnp.bfloat16)
    expbig = jnp.tile(exp8, (1, L2))

    sel = jnp.pad(eyec, ((0, (L2 - 1) * C), (0, 0)))

    xst = x_src.transpose(1, 2, 0).reshape(L2 * C, n)
    xdt = x_dst.transpose(1, 2, 0).reshape(L2 * C, n)
    tsa, tsb = _run_table_kernel(xst, w1s_bd8, sel)
    tda, tdb = _run_table_kernel(xdt, w1d_bd8, sel)

    z = jnp.zeros((N_PAD, SCW), f32)
    half1 = (e // 2) // EB * EB
    all_nodes = []
    for e0, ecnt in ((0, half1), (half1, e - half1)):
        esl = jax.lax.slice(edge_src, (e0,), (e0 + ecnt,))
        edl = jax.lax.slice(edge_dst, (e0,), (e0 + ecnt,))

        erows = ecnt // GCH
        g_base = ((erows + SC_NW - 1) // SC_NW + 7) // 8 * 8
        es2 = jnp.pad(esl.reshape(erows, GCH),
                      ((0, g_base * SC_NW - erows), (0, 0)))
        ed2 = jnp.pad(edl.reshape(erows, GCH),
                      ((0, g_base * SC_NW - erows), (0, 0)))
        gas, gad, gbs, gbd = _sc_gather_make(n, ecnt)(tsa, tda, tsb, tdb,
                                                      es2, ed2)

        vals = _run_edge_kernel(
            edge_distance.T, gas, gad, gbs, gbd, Wr1, br1.reshape(1, -1), Wr2,
            br2.reshape(1, -1), W0, b0.reshape(1, -1), lng, lnb, proj, ad, g8,
            w2_bd, expbig, off=e0 // EB)

        srows = ecnt // SCH
        s_base = ((srows + SC_NS - 1) // SC_NS + 7) // 8 * 8
        eds2 = jnp.pad(edl.reshape(srows, SCH),
                       ((0, s_base * SC_NS - srows + 64), (0, 0)))
        all_nodes.extend(_sc_scatter_make(ecnt, N_PAD)(*vals, eds2, z))

    out = _run_node_kernel(all_nodes, Wp, bp.reshape(1, -1), exp8)
    return out[:n].reshape(n, L2, OUT)
